# Initial kernel scaffold; baseline (speedup 1.0000x reference)
import jax
import jax.numpy as jnp
from jax.experimental import pallas as pl


def kernel(x, edge_index, edge_type, in_W, in_b, bases, comp, root_w, conv_b, ln_g, ln_b, hW1, hb1, hW2, hb2):
    raise NotImplementedError("write your pallas kernel here")


# speedup vs baseline: 8.1589x; 8.1589x over previous
import functools

import jax
import jax.numpy as jnp
from jax import lax
from jax.experimental import pallas as pl
from jax.experimental.pallas import tpu as pltpu
from jax.experimental.pallas import tpu_sc as plsc

N = 10000
E = 160000
D = 128
H = 128
OUT = 64
R = 10
NB = 4
L = 3

NP = 10240
R1 = R + 1
CNTP = R * NP
NC = 2
NS = 16
NW = NC * NS
EPT = 5120
CHUNK = 128
NCH = EPT // CHUNK
EP = NW * EPT
EROWS = EP // CHUNK
CROWS = EP // NS // CHUNK
NROWS_W = NP // NS // CHUNK
CPT = CNTP // NS

BM = 256
NBLK = NP // BM

_MESH = plsc.VectorSubcoreMesh(
    core_axis_name="c", subcore_axis_name="s", num_cores=NC, num_subcores=NS)


def _zero_rows(ref, nrows):
    def body(i, _):
        for q in range(CHUNK // 16):
            ref[i, pl.ds(q * 16, 16)] = jnp.zeros((16,), jnp.float32)
        return 0
    lax.fori_loop(0, nrows, body, 0, unroll=False)


@functools.partial(
    pl.kernel,
    out_type=jax.ShapeDtypeStruct((EP,), jnp.float32),
    mesh=_MESH,
    scratch_types=[
        pltpu.VMEM((CROWS, CHUNK), jnp.int32),
        pltpu.VMEM((EP // NS,), jnp.float32),
        pltpu.VMEM((NCH, CHUNK), jnp.int32),
        pltpu.VMEM((EPT,), jnp.float32),
        pltpu.VMEM((EPT,), jnp.float32),
        pltpu.VMEM((EPT,), jnp.float32),
        pltpu.VMEM((CPT,), jnp.float32),
        pltpu.VMEM_SHARED((CNTP,), jnp.float32),
    ],
)
def _count_weights(key_hbm, valid_hbm, w_hbm,
                   key_a, valid_a, key_b, valid_b, cnt_g, w_v, zbuf, cnt_sh):
    c = lax.axis_index("c")
    s = lax.axis_index("s")
    wid = c * NS + s
    def zbody(i, _):
        zbuf[pl.ds(i * 16, 16)] = jnp.zeros((16,), jnp.float32)
        return 0
    lax.fori_loop(0, CPT // 16, zbody, 0, unroll=False)
    pltpu.sync_copy(zbuf, cnt_sh.at[pl.ds(s * CPT, CPT)])
    plsc.subcore_barrier()
    pltpu.sync_copy(key_hbm.at[pl.ds(s * CROWS, CROWS)], key_a)
    pltpu.sync_copy(valid_hbm.at[pl.ds(s * (EP // NS), EP // NS)], valid_a)

    def cbody(j, _):
        pltpu.sync_copy(valid_a.at[pl.ds(j * CHUNK, CHUNK)],
                        cnt_sh.at[key_a.at[j]], add=True)
        return 0
    lax.fori_loop(0, CROWS, cbody, 0, unroll=False)
    plsc.subcore_barrier()
    pltpu.sync_copy(key_hbm.at[pl.ds(wid * NCH, NCH)], key_b)
    pltpu.sync_copy(valid_hbm.at[pl.ds(wid * EPT, EPT)], valid_b)

    def gbody(j, _):
        pltpu.sync_copy(cnt_sh.at[key_b.at[j]],
                        cnt_g.at[pl.ds(j * CHUNK, CHUNK)])
        return 0
    lax.fori_loop(0, NCH, gbody, 0, unroll=False)

    def wbody(i, _):
        sl = pl.ds(i * 16, 16)
        w_v[sl] = valid_b[sl] / jnp.maximum(cnt_g[sl], 1.0)
        return 0
    lax.fori_loop(0, EPT // 16, wbody, 0, unroll=False)
    pltpu.sync_copy(w_v, w_hbm.at[pl.ds(wid * EPT, EPT)])


@functools.partial(
    pl.kernel,
    out_type=jax.ShapeDtypeStruct((NC, NP, H), jnp.float32),
    mesh=_MESH,
    scratch_types=[
        pltpu.VMEM((NCH, CHUNK), jnp.int32),
        pltpu.VMEM((NCH, CHUNK), jnp.int32),
        pltpu.VMEM((EPT,), jnp.float32),
        pltpu.VMEM((CHUNK, H), jnp.float32),
        pltpu.VMEM_SHARED((NP, H), jnp.float32),
        pltpu.SemaphoreType.DMA,
    ],
    compiler_params=pltpu.CompilerParams(needs_layout_passes=False),
)
def _aggregate(xr_hbm, g_hbm, dst_hbm, w_hbm, out_hbm,
               g_v, dst_v, w_v, rows, acc_sh, sem):
    c = lax.axis_index("c")
    s = lax.axis_index("s")
    wid = c * NS + s
    _zero_rows(rows, CHUNK)
    for t in range(NROWS_W):
        pltpu.sync_copy(rows, acc_sh.at[pl.ds((s * NROWS_W + t) * CHUNK, CHUNK)])
    plsc.subcore_barrier()
    pltpu.sync_copy(g_hbm.at[pl.ds(wid * NCH, NCH)], g_v)
    pltpu.sync_copy(dst_hbm.at[pl.ds(wid * NCH, NCH)], dst_v)
    pltpu.sync_copy(w_hbm.at[pl.ds(wid * EPT, EPT)], w_v)

    def chunk_body(j, _):
        pltpu.async_copy(xr_hbm.at[g_v.at[j]], rows, sem).wait()
        jbase = j * CHUNK

        def row_body(i, _):
            ws = plsc.load_gather(w_v, [jnp.full((16,), jbase + i, jnp.int32)])
            for q in range(H // 16):
                sl = pl.ds(q * 16, 16)
                rows[i, sl] = rows[i, sl] * ws
            return 0
        lax.fori_loop(0, CHUNK, row_body, 0, unroll=False)
        pltpu.sync_copy(rows, acc_sh.at[dst_v.at[j]], add=True)
        return 0
    lax.fori_loop(0, NCH, chunk_body, 0, unroll=False)
    plsc.subcore_barrier()
    for t in range(NROWS_W):
        rs = pl.ds((s * NROWS_W + t) * CHUNK, CHUNK)
        pltpu.sync_copy(acc_sh.at[rs], rows)
        pltpu.sync_copy(rows, out_hbm.at[c, rs])


def _combine_weights(bases, comp, root_w):
    def body(comp_ref, bases_ref, root_ref, out_ref):
        for l in range(L):
            for r in range(R):
                acc = comp_ref[l, r, 0] * bases_ref[l, 0]
                for b in range(1, NB):
                    acc = acc + comp_ref[l, r, b] * bases_ref[l, b]
                out_ref[l, r] = acc
            out_ref[l, R] = root_ref[l]
    return pl.pallas_call(
        body,
        out_shape=jax.ShapeDtypeStruct((L, R1, H, H), jnp.float32),
        in_specs=[
            pl.BlockSpec(memory_space=pltpu.SMEM),
            pl.BlockSpec(memory_space=pltpu.VMEM),
            pl.BlockSpec(memory_space=pltpu.VMEM),
        ],
        out_specs=pl.BlockSpec(memory_space=pltpu.VMEM),
    )(comp, bases, root_w)


def _input_proj(x_pad, in_W, in_b):
    def body(x_ref, w_ref, b_ref, out_ref):
        out_ref[...] = jnp.dot(x_ref[...], w_ref[...],
                               preferred_element_type=jnp.float32) + b_ref[...]
    return pl.pallas_call(
        body,
        grid=(NBLK,),
        out_shape=jax.ShapeDtypeStruct((NP, H), jnp.float32),
        in_specs=[
            pl.BlockSpec((BM, D), lambda n: (n, 0)),
            pl.BlockSpec((D, H), lambda n: (0, 0)),
            pl.BlockSpec((1, H), lambda n: (0, 0)),
        ],
        out_specs=pl.BlockSpec((BM, H), lambda n: (n, 0)),
    )(x_pad, in_W, in_b)


def _xr_all(h, Waug, l):
    def body(h_ref, w_ref, out_ref):
        out_ref[...] = jnp.dot(h_ref[...], w_ref[0, 0],
                               preferred_element_type=jnp.float32)
    return pl.pallas_call(
        body,
        grid=(R1, NBLK),
        out_shape=jax.ShapeDtypeStruct((R1 * NP, H), jnp.float32),
        in_specs=[
            pl.BlockSpec((BM, H), lambda r, n: (n, 0)),
            pl.BlockSpec((1, 1, H, H), lambda r, n: (l, r, 0, 0)),
        ],
        out_specs=pl.BlockSpec((BM, H), lambda r, n: (r * NBLK + n, 0)),
    )(h, Waug)


def _post(S, xr_flat, h_prev, conv_b, ln_g, ln_b, l, residual):
    def body(s_ref, xr_ref, h_ref, cb_ref, g_ref, b_ref, out_ref):
        t = xr_ref[...] + s_ref[0] + s_ref[1] + cb_ref[0]
        m = jnp.mean(t, axis=1, keepdims=True)
        d = t - m
        v = jnp.mean(d * d, axis=1, keepdims=True)
        hn = d * lax.rsqrt(v + 1e-5) * g_ref[0] + b_ref[0]
        hn = jnp.maximum(hn, 0.0)
        if residual:
            hn = hn + h_ref[...]
        out_ref[...] = hn
    return pl.pallas_call(
        body,
        grid=(NBLK,),
        out_shape=jax.ShapeDtypeStruct((NP, H), jnp.float32),
        in_specs=[
            pl.BlockSpec((NC, BM, H), lambda n: (0, n, 0)),
            pl.BlockSpec((BM, H), lambda n: (R * NBLK + n, 0)),
            pl.BlockSpec((BM, H), lambda n: (n, 0)),
            pl.BlockSpec((1, 1, H), lambda n: (l, 0, 0)),
            pl.BlockSpec((1, 1, H), lambda n: (l, 0, 0)),
            pl.BlockSpec((1, 1, H), lambda n: (l, 0, 0)),
        ],
        out_specs=pl.BlockSpec((BM, H), lambda n: (n, 0)),
    )(S, xr_flat, h_prev, conv_b, ln_g, ln_b)


def _mlp(h, hW1, hb1, hW2, hb2):
    def body(h_ref, w1_ref, b1_ref, w2_ref, b2_ref, out_ref):
        h2 = jnp.dot(h_ref[...], w1_ref[...],
                     preferred_element_type=jnp.float32) + b1_ref[...]
        h2 = jnp.maximum(h2, 0.0)
        out_ref[...] = jnp.dot(h2, w2_ref[...],
                               preferred_element_type=jnp.float32) + b2_ref[...]
    return pl.pallas_call(
        body,
        grid=(NBLK,),
        out_shape=jax.ShapeDtypeStruct((NP, OUT), jnp.float32),
        in_specs=[
            pl.BlockSpec((BM, H), lambda n: (n, 0)),
            pl.BlockSpec((H, H), lambda n: (0, 0)),
            pl.BlockSpec((1, H), lambda n: (0, 0)),
            pl.BlockSpec((H, OUT), lambda n: (0, 0)),
            pl.BlockSpec((1, OUT), lambda n: (0, 0)),
        ],
        out_specs=pl.BlockSpec((BM, OUT), lambda n: (n, 0)),
    )(h, hW1, hb1, hW2, hb2)


def _edge_layout(a):
    a = a.reshape(NW, E // NW)
    a = jnp.pad(a, ((0, 0), (0, EPT - E // NW)))
    return a.reshape(EROWS, CHUNK)


def kernel(x, edge_index, edge_type, in_W, in_b, bases, comp, root_w,
           conv_b, ln_g, ln_b, hW1, hb1, hW2, hb2):
    src = edge_index[0]
    dst = edge_index[1]
    et = edge_type
    g2d = _edge_layout(et * NP + src)
    dst2d = _edge_layout(dst)
    key2d = _edge_layout(et * NP + dst)
    valid = jnp.ones((NW, E // NW), jnp.float32)
    valid1d = jnp.pad(valid, ((0, 0), (0, EPT - E // NW))).reshape(EP)

    w1d = _count_weights(key2d, valid1d)

    Waug = _combine_weights(bases, comp, root_w)
    x_pad = jnp.pad(x, ((0, NP - N), (0, 0)))
    h = _input_proj(x_pad, in_W, in_b.reshape(1, H))

    cb3 = conv_b.reshape(L, 1, H)
    g3 = ln_g.reshape(L, 1, H)
    b3 = ln_b.reshape(L, 1, H)
    for l in range(L):
        xr_flat = _xr_all(h, Waug, l)
        S = _aggregate(xr_flat, g2d, dst2d, w1d)
        h = _post(S, xr_flat, h, cb3, g3, b3, l, residual=(l > 0))

    out = _mlp(h, hW1, hb1.reshape(1, H), hW2, hb2.reshape(1, OUT))
    return out[:N]


# speedup vs baseline: 8.9606x; 1.0983x over previous
import functools

import jax
import jax.numpy as jnp
from jax import lax
from jax.experimental import pallas as pl
from jax.experimental.pallas import tpu as pltpu
from jax.experimental.pallas import tpu_sc as plsc

N = 10000
E = 160000
D = 128
H = 128
OUT = 64
R = 10
NB = 4
L = 3

NP = 10240
R1 = R + 1
CNTP = R * NP
NC = 2
NS = 16
NW = NC * NS
EPT = 5120
CHUNK = 128
NCH = EPT // CHUNK
EP = NW * EPT
EROWS = EP // CHUNK
CROWS = EP // NS // CHUNK
NROWS_W = NP // NS // CHUNK
CPT = CNTP // NS

BM = 256
NBLK = NP // BM

_MESH = plsc.VectorSubcoreMesh(
    core_axis_name="c", subcore_axis_name="s", num_cores=NC, num_subcores=NS)


def _zero_rows(ref, nrows):
    def body(i, _):
        for q in range(CHUNK // 16):
            ref[i, pl.ds(q * 16, 16)] = jnp.zeros((16,), jnp.float32)
        return 0
    lax.fori_loop(0, nrows, body, 0, unroll=False)


@functools.partial(
    pl.kernel,
    out_type=jax.ShapeDtypeStruct((EP,), jnp.float32),
    mesh=_MESH,
    scratch_types=[
        pltpu.VMEM((CROWS, CHUNK), jnp.int32),
        pltpu.VMEM((EP // NS,), jnp.float32),
        pltpu.VMEM((NCH, CHUNK), jnp.int32),
        pltpu.VMEM((EPT,), jnp.float32),
        pltpu.VMEM((EPT,), jnp.float32),
        pltpu.VMEM((EPT,), jnp.float32),
        pltpu.VMEM((CPT,), jnp.float32),
        pltpu.VMEM_SHARED((CNTP,), jnp.float32),
    ],
)
def _count_weights(key_hbm, valid_hbm, w_hbm,
                   key_a, valid_a, key_b, valid_b, cnt_g, w_v, zbuf, cnt_sh):
    c = lax.axis_index("c")
    s = lax.axis_index("s")
    wid = c * NS + s
    def zbody(i, _):
        zbuf[pl.ds(i * 16, 16)] = jnp.zeros((16,), jnp.float32)
        return 0
    lax.fori_loop(0, CPT // 16, zbody, 0, unroll=False)
    pltpu.sync_copy(zbuf, cnt_sh.at[pl.ds(s * CPT, CPT)])
    plsc.subcore_barrier()
    pltpu.sync_copy(key_hbm.at[pl.ds(s * CROWS, CROWS)], key_a)
    pltpu.sync_copy(valid_hbm.at[pl.ds(s * (EP // NS), EP // NS)], valid_a)

    def cbody(j, _):
        pltpu.sync_copy(valid_a.at[pl.ds(j * CHUNK, CHUNK)],
                        cnt_sh.at[key_a.at[j]], add=True)
        return 0
    lax.fori_loop(0, CROWS, cbody, 0, unroll=False)
    plsc.subcore_barrier()
    pltpu.sync_copy(key_hbm.at[pl.ds(wid * NCH, NCH)], key_b)
    pltpu.sync_copy(valid_hbm.at[pl.ds(wid * EPT, EPT)], valid_b)

    def gbody(j, _):
        pltpu.sync_copy(cnt_sh.at[key_b.at[j]],
                        cnt_g.at[pl.ds(j * CHUNK, CHUNK)])
        return 0
    lax.fori_loop(0, NCH, gbody, 0, unroll=False)

    def wbody(i, _):
        sl = pl.ds(i * 16, 16)
        w_v[sl] = valid_b[sl] / jnp.maximum(cnt_g[sl], 1.0)
        return 0
    lax.fori_loop(0, EPT // 16, wbody, 0, unroll=False)
    pltpu.sync_copy(w_v, w_hbm.at[pl.ds(wid * EPT, EPT)])


@functools.partial(
    pl.kernel,
    out_type=jax.ShapeDtypeStruct((NC, NP, H), jnp.float32),
    mesh=_MESH,
    scratch_types=[
        pltpu.VMEM((NCH, CHUNK), jnp.int32),
        pltpu.VMEM((NCH, CHUNK), jnp.int32),
        pltpu.VMEM((EPT,), jnp.float32),
        pltpu.VMEM((CHUNK, H), jnp.float32),
        pltpu.VMEM((CHUNK, H), jnp.float32),
        pltpu.VMEM_SHARED((NP, H), jnp.float32),
        pltpu.SemaphoreType.DMA,
    ],
    compiler_params=pltpu.CompilerParams(needs_layout_passes=False),
)
def _aggregate(xr_hbm, g_hbm, dst_hbm, w_hbm, out_hbm,
               g_v, dst_v, w_v, rows0, rows1, acc_sh, sem):
    c = lax.axis_index("c")
    s = lax.axis_index("s")
    wid = c * NS + s
    _zero_rows(rows0, CHUNK)
    for t in range(NROWS_W):
        pltpu.sync_copy(rows0, acc_sh.at[pl.ds((s * NROWS_W + t) * CHUNK, CHUNK)])
    plsc.subcore_barrier()
    pltpu.sync_copy(g_hbm.at[pl.ds(wid * NCH, NCH)], g_v)
    pltpu.sync_copy(dst_hbm.at[pl.ds(wid * NCH, NCH)], dst_v)
    pltpu.sync_copy(w_hbm.at[pl.ds(wid * EPT, EPT)], w_v)

    bufs = (rows0, rows1)
    pltpu.async_copy(xr_hbm.at[g_v.at[0]], rows0, sem)

    def scale_and_scatter(j, buf):
        jbase = j * CHUNK

        def row_body(i, _):
            ws = plsc.load_gather(w_v, [jnp.full((16,), jbase + i, jnp.int32)])
            for q in range(H // 16):
                sl = pl.ds(q * 16, 16)
                buf[i, sl] = buf[i, sl] * ws
            return 0
        lax.fori_loop(0, CHUNK, row_body, 0, unroll=4)
        pltpu.sync_copy(buf, acc_sh.at[dst_v.at[j]], add=True)

    def pair_body(j2, _):
        for b in range(2):
            j = j2 * 2 + b
            cur, nxt = bufs[b], bufs[1 - b]

            @pl.when(j < NCH - 1)
            def _():
                pltpu.async_copy(xr_hbm.at[g_v.at[j + 1]], nxt, sem)
            pltpu.make_async_copy(xr_hbm.at[g_v.at[j]], cur, sem).wait()
            scale_and_scatter(j, cur)
        return 0
    lax.fori_loop(0, NCH // 2, pair_body, 0, unroll=False)
    plsc.subcore_barrier()
    for t in range(NROWS_W):
        rs = pl.ds((s * NROWS_W + t) * CHUNK, CHUNK)
        pltpu.sync_copy(acc_sh.at[rs], rows0)
        pltpu.sync_copy(rows0, out_hbm.at[c, rs])


def _combine_weights(bases, comp, root_w):
    def body(comp_ref, bases_ref, root_ref, out_ref):
        for l in range(L):
            for r in range(R):
                acc = comp_ref[l, r, 0] * bases_ref[l, 0]
                for b in range(1, NB):
                    acc = acc + comp_ref[l, r, b] * bases_ref[l, b]
                out_ref[l, r] = acc
            out_ref[l, R] = root_ref[l]
    return pl.pallas_call(
        body,
        out_shape=jax.ShapeDtypeStruct((L, R1, H, H), jnp.float32),
        in_specs=[
            pl.BlockSpec(memory_space=pltpu.SMEM),
            pl.BlockSpec(memory_space=pltpu.VMEM),
            pl.BlockSpec(memory_space=pltpu.VMEM),
        ],
        out_specs=pl.BlockSpec(memory_space=pltpu.VMEM),
    )(comp, bases, root_w)


def _input_proj(x_pad, in_W, in_b):
    def body(x_ref, w_ref, b_ref, out_ref):
        out_ref[...] = jnp.dot(x_ref[...], w_ref[...],
                               preferred_element_type=jnp.float32) + b_ref[...]
    return pl.pallas_call(
        body,
        grid=(NBLK,),
        out_shape=jax.ShapeDtypeStruct((NP, H), jnp.float32),
        in_specs=[
            pl.BlockSpec((BM, D), lambda n: (n, 0)),
            pl.BlockSpec((D, H), lambda n: (0, 0)),
            pl.BlockSpec((1, H), lambda n: (0, 0)),
        ],
        out_specs=pl.BlockSpec((BM, H), lambda n: (n, 0)),
    )(x_pad, in_W, in_b)


def _xr_all(h, Waug, l):
    def body(h_ref, w_ref, out_ref):
        out_ref[...] = jnp.dot(h_ref[...], w_ref[0, 0],
                               preferred_element_type=jnp.float32)
    return pl.pallas_call(
        body,
        grid=(R1, NBLK),
        out_shape=jax.ShapeDtypeStruct((R1 * NP, H), jnp.float32),
        in_specs=[
            pl.BlockSpec((BM, H), lambda r, n: (n, 0)),
            pl.BlockSpec((1, 1, H, H), lambda r, n: (l, r, 0, 0)),
        ],
        out_specs=pl.BlockSpec((BM, H), lambda r, n: (r * NBLK + n, 0)),
    )(h, Waug)


def _post(S, xr_flat, h_prev, conv_b, ln_g, ln_b, l, residual):
    def body(s_ref, xr_ref, h_ref, cb_ref, g_ref, b_ref, out_ref):
        t = xr_ref[...] + s_ref[0] + s_ref[1] + cb_ref[0]
        m = jnp.mean(t, axis=1, keepdims=True)
        d = t - m
        v = jnp.mean(d * d, axis=1, keepdims=True)
        hn = d * lax.rsqrt(v + 1e-5) * g_ref[0] + b_ref[0]
        hn = jnp.maximum(hn, 0.0)
        if residual:
            hn = hn + h_ref[...]
        out_ref[...] = hn
    return pl.pallas_call(
        body,
        grid=(NBLK,),
        out_shape=jax.ShapeDtypeStruct((NP, H), jnp.float32),
        in_specs=[
            pl.BlockSpec((NC, BM, H), lambda n: (0, n, 0)),
            pl.BlockSpec((BM, H), lambda n: (R * NBLK + n, 0)),
            pl.BlockSpec((BM, H), lambda n: (n, 0)),
            pl.BlockSpec((1, 1, H), lambda n: (l, 0, 0)),
            pl.BlockSpec((1, 1, H), lambda n: (l, 0, 0)),
            pl.BlockSpec((1, 1, H), lambda n: (l, 0, 0)),
        ],
        out_specs=pl.BlockSpec((BM, H), lambda n: (n, 0)),
    )(S, xr_flat, h_prev, conv_b, ln_g, ln_b)


def _mlp(h, hW1, hb1, hW2, hb2):
    def body(h_ref, w1_ref, b1_ref, w2_ref, b2_ref, out_ref):
        h2 = jnp.dot(h_ref[...], w1_ref[...],
                     preferred_element_type=jnp.float32) + b1_ref[...]
        h2 = jnp.maximum(h2, 0.0)
        out_ref[...] = jnp.dot(h2, w2_ref[...],
                               preferred_element_type=jnp.float32) + b2_ref[...]
    return pl.pallas_call(
        body,
        grid=(NBLK,),
        out_shape=jax.ShapeDtypeStruct((NP, OUT), jnp.float32),
        in_specs=[
            pl.BlockSpec((BM, H), lambda n: (n, 0)),
            pl.BlockSpec((H, H), lambda n: (0, 0)),
            pl.BlockSpec((1, H), lambda n: (0, 0)),
            pl.BlockSpec((H, OUT), lambda n: (0, 0)),
            pl.BlockSpec((1, OUT), lambda n: (0, 0)),
        ],
        out_specs=pl.BlockSpec((BM, OUT), lambda n: (n, 0)),
    )(h, hW1, hb1, hW2, hb2)


def _edge_layout(a):
    a = a.reshape(NW, E // NW)
    a = jnp.pad(a, ((0, 0), (0, EPT - E // NW)))
    return a.reshape(EROWS, CHUNK)


def kernel(x, edge_index, edge_type, in_W, in_b, bases, comp, root_w,
           conv_b, ln_g, ln_b, hW1, hb1, hW2, hb2):
    src = edge_index[0]
    dst = edge_index[1]
    et = edge_type
    g2d = _edge_layout(et * NP + src)
    dst2d = _edge_layout(dst)
    key2d = _edge_layout(et * NP + dst)
    valid = jnp.ones((NW, E // NW), jnp.float32)
    valid1d = jnp.pad(valid, ((0, 0), (0, EPT - E // NW))).reshape(EP)

    w1d = _count_weights(key2d, valid1d)

    Waug = _combine_weights(bases, comp, root_w)
    x_pad = jnp.pad(x, ((0, NP - N), (0, 0)))
    h = _input_proj(x_pad, in_W, in_b.reshape(1, H))

    cb3 = conv_b.reshape(L, 1, H)
    g3 = ln_g.reshape(L, 1, H)
    b3 = ln_b.reshape(L, 1, H)
    for l in range(L):
        xr_flat = _xr_all(h, Waug, l)
        S = _aggregate(xr_flat, g2d, dst2d, w1d)
        h = _post(S, xr_flat, h, cb3, g3, b3, l, residual=(l > 0))

    out = _mlp(h, hW1, hb1.reshape(1, H), hW2, hb2.reshape(1, OUT))
    return out[:N]


# speedup vs baseline: 9.2162x; 1.0285x over previous
import functools

import jax
import jax.numpy as jnp
from jax import lax
from jax.experimental import pallas as pl
from jax.experimental.pallas import tpu as pltpu
from jax.experimental.pallas import tpu_sc as plsc

N = 10000
E = 160000
D = 128
H = 128
OUT = 64
R = 10
NB = 4
L = 3

NP = 10240
R1 = R + 1
CNTP = R * NP
NC = 2
NS = 16
NW = NC * NS
EPT = 5120
CHUNK = 128
NCH = EPT // CHUNK
EP = NW * EPT
EROWS = EP // CHUNK
CROWS = EP // NS // CHUNK
NROWS_W = NP // NS // CHUNK
CPT = CNTP // NS

BM = 256
NBLK = NP // BM

_MESH = plsc.VectorSubcoreMesh(
    core_axis_name="c", subcore_axis_name="s", num_cores=NC, num_subcores=NS)


def _zero_rows(ref, nrows):
    def body(i, _):
        for q in range(CHUNK // 16):
            ref[i, pl.ds(q * 16, 16)] = jnp.zeros((16,), jnp.float32)
        return 0
    lax.fori_loop(0, nrows, body, 0, unroll=False)


@functools.partial(
    pl.kernel,
    out_type=jax.ShapeDtypeStruct((EP,), jnp.float32),
    mesh=_MESH,
    scratch_types=[
        pltpu.VMEM((CROWS, CHUNK), jnp.int32),
        pltpu.VMEM((EP // NS,), jnp.float32),
        pltpu.VMEM((NCH, CHUNK), jnp.int32),
        pltpu.VMEM((EPT,), jnp.float32),
        pltpu.VMEM((EPT,), jnp.float32),
        pltpu.VMEM((EPT,), jnp.float32),
        pltpu.VMEM((CPT,), jnp.float32),
        pltpu.VMEM_SHARED((CNTP,), jnp.float32),
    ],
)
def _count_weights(key_hbm, valid_hbm, w_hbm,
                   key_a, valid_a, key_b, valid_b, cnt_g, w_v, zbuf, cnt_sh):
    c = lax.axis_index("c")
    s = lax.axis_index("s")
    wid = c * NS + s
    def zbody(i, _):
        zbuf[pl.ds(i * 16, 16)] = jnp.zeros((16,), jnp.float32)
        return 0
    lax.fori_loop(0, CPT // 16, zbody, 0, unroll=False)
    pltpu.sync_copy(zbuf, cnt_sh.at[pl.ds(s * CPT, CPT)])
    plsc.subcore_barrier()
    pltpu.sync_copy(key_hbm.at[pl.ds(s * CROWS, CROWS)], key_a)
    pltpu.sync_copy(valid_hbm.at[pl.ds(s * (EP // NS), EP // NS)], valid_a)

    def cbody(j, _):
        pltpu.sync_copy(valid_a.at[pl.ds(j * CHUNK, CHUNK)],
                        cnt_sh.at[key_a.at[j]], add=True)
        return 0
    lax.fori_loop(0, CROWS, cbody, 0, unroll=False)
    plsc.subcore_barrier()
    pltpu.sync_copy(key_hbm.at[pl.ds(wid * NCH, NCH)], key_b)
    pltpu.sync_copy(valid_hbm.at[pl.ds(wid * EPT, EPT)], valid_b)

    def gbody(j, _):
        pltpu.sync_copy(cnt_sh.at[key_b.at[j]],
                        cnt_g.at[pl.ds(j * CHUNK, CHUNK)])
        return 0
    lax.fori_loop(0, NCH, gbody, 0, unroll=False)

    def wbody(i, _):
        sl = pl.ds(i * 16, 16)
        w_v[sl] = valid_b[sl] / jnp.maximum(cnt_g[sl], 1.0)
        return 0
    lax.fori_loop(0, EPT // 16, wbody, 0, unroll=False)
    pltpu.sync_copy(w_v, w_hbm.at[pl.ds(wid * EPT, EPT)])


@functools.partial(
    pl.kernel,
    out_type=jax.ShapeDtypeStruct((NC, NP, H), jnp.float32),
    mesh=_MESH,
    scratch_types=[
        pltpu.VMEM((NCH, CHUNK), jnp.int32),
        pltpu.VMEM((NCH, CHUNK), jnp.int32),
        pltpu.VMEM((EPT,), jnp.float32),
        pltpu.VMEM((CHUNK, H), jnp.float32),
        pltpu.VMEM((CHUNK, H), jnp.float32),
        pltpu.VMEM_SHARED((NP, H), jnp.float32),
        pltpu.SemaphoreType.DMA,
        pltpu.SemaphoreType.DMA,
    ],
    compiler_params=pltpu.CompilerParams(needs_layout_passes=False),
)
def _aggregate(xr_hbm, g_hbm, dst_hbm, w_hbm, out_hbm,
               g_v, dst_v, w_v, rows0, rows1, acc_sh, sem, sem_s):
    c = lax.axis_index("c")
    s = lax.axis_index("s")
    wid = c * NS + s
    _zero_rows(rows0, CHUNK)
    for t in range(NROWS_W):
        pltpu.sync_copy(rows0, acc_sh.at[pl.ds((s * NROWS_W + t) * CHUNK, CHUNK)])
    plsc.subcore_barrier()
    pltpu.sync_copy(g_hbm.at[pl.ds(wid * NCH, NCH)], g_v)
    pltpu.sync_copy(dst_hbm.at[pl.ds(wid * NCH, NCH)], dst_v)
    pltpu.sync_copy(w_hbm.at[pl.ds(wid * EPT, EPT)], w_v)

    bufs = (rows0, rows1)
    pltpu.async_copy(xr_hbm.at[g_v.at[0]], rows0, sem)

    def scale(j, buf):
        jbase = j * CHUNK

        def grp_body(t, _):
            wg = w_v[pl.ds(jbase + t * 16, 16)]
            for r in range(16):
                ws = wg[r]
                for q in range(H // 16):
                    sl = pl.ds(q * 16, 16)
                    buf[t * 16 + r, sl] = buf[t * 16 + r, sl] * ws
            return 0
        lax.fori_loop(0, CHUNK // 16, grp_body, 0, unroll=False)

    def pair_body(j2, _):
        for b in range(2):
            j = j2 * 2 + b
            cur, nxt = bufs[b], bufs[1 - b]

            @pl.when(j >= 1)
            def _():
                pltpu.make_async_copy(
                    nxt, acc_sh.at[dst_v.at[j - 1]], sem_s).wait()

            @pl.when(j < NCH - 1)
            def _():
                pltpu.async_copy(xr_hbm.at[g_v.at[j + 1]], nxt, sem)
            pltpu.make_async_copy(xr_hbm.at[g_v.at[j]], cur, sem).wait()
            scale(j, cur)
            pltpu.async_copy(cur, acc_sh.at[dst_v.at[j]], sem_s, add=True)
        return 0
    lax.fori_loop(0, NCH // 2, pair_body, 0, unroll=False)
    pltpu.make_async_copy(
        bufs[1], acc_sh.at[dst_v.at[NCH - 1]], sem_s).wait()
    plsc.subcore_barrier()
    for t in range(NROWS_W):
        rs = pl.ds((s * NROWS_W + t) * CHUNK, CHUNK)
        pltpu.sync_copy(acc_sh.at[rs], rows0)
        pltpu.sync_copy(rows0, out_hbm.at[c, rs])


def _combine_weights(bases, comp, root_w):
    def body(comp_ref, bases_ref, root_ref, out_ref):
        for l in range(L):
            for r in range(R):
                acc = comp_ref[l, r, 0] * bases_ref[l, 0]
                for b in range(1, NB):
                    acc = acc + comp_ref[l, r, b] * bases_ref[l, b]
                out_ref[l, r] = acc
            out_ref[l, R] = root_ref[l]
    return pl.pallas_call(
        body,
        out_shape=jax.ShapeDtypeStruct((L, R1, H, H), jnp.float32),
        in_specs=[
            pl.BlockSpec(memory_space=pltpu.SMEM),
            pl.BlockSpec(memory_space=pltpu.VMEM),
            pl.BlockSpec(memory_space=pltpu.VMEM),
        ],
        out_specs=pl.BlockSpec(memory_space=pltpu.VMEM),
    )(comp, bases, root_w)


def _input_proj(x_pad, in_W, in_b):
    def body(x_ref, w_ref, b_ref, out_ref):
        out_ref[...] = jnp.dot(x_ref[...], w_ref[...],
                               preferred_element_type=jnp.float32) + b_ref[...]
    return pl.pallas_call(
        body,
        grid=(NBLK,),
        out_shape=jax.ShapeDtypeStruct((NP, H), jnp.float32),
        in_specs=[
            pl.BlockSpec((BM, D), lambda n: (n, 0)),
            pl.BlockSpec((D, H), lambda n: (0, 0)),
            pl.BlockSpec((1, H), lambda n: (0, 0)),
        ],
        out_specs=pl.BlockSpec((BM, H), lambda n: (n, 0)),
    )(x_pad, in_W, in_b)


def _xr_all(h, Waug, l):
    def body(h_ref, w_ref, out_ref):
        out_ref[...] = jnp.dot(h_ref[...], w_ref[0, 0],
                               preferred_element_type=jnp.float32)
    return pl.pallas_call(
        body,
        grid=(R1, NBLK),
        out_shape=jax.ShapeDtypeStruct((R1 * NP, H), jnp.float32),
        in_specs=[
            pl.BlockSpec((BM, H), lambda r, n: (n, 0)),
            pl.BlockSpec((1, 1, H, H), lambda r, n: (l, r, 0, 0)),
        ],
        out_specs=pl.BlockSpec((BM, H), lambda r, n: (r * NBLK + n, 0)),
    )(h, Waug)


def _post(S, xr_flat, h_prev, conv_b, ln_g, ln_b, l, residual):
    def body(s_ref, xr_ref, h_ref, cb_ref, g_ref, b_ref, out_ref):
        t = xr_ref[...] + s_ref[0] + s_ref[1] + cb_ref[0]
        m = jnp.mean(t, axis=1, keepdims=True)
        d = t - m
        v = jnp.mean(d * d, axis=1, keepdims=True)
        hn = d * lax.rsqrt(v + 1e-5) * g_ref[0] + b_ref[0]
        hn = jnp.maximum(hn, 0.0)
        if residual:
            hn = hn + h_ref[...]
        out_ref[...] = hn
    return pl.pallas_call(
        body,
        grid=(NBLK,),
        out_shape=jax.ShapeDtypeStruct((NP, H), jnp.float32),
        in_specs=[
            pl.BlockSpec((NC, BM, H), lambda n: (0, n, 0)),
            pl.BlockSpec((BM, H), lambda n: (R * NBLK + n, 0)),
            pl.BlockSpec((BM, H), lambda n: (n, 0)),
            pl.BlockSpec((1, 1, H), lambda n: (l, 0, 0)),
            pl.BlockSpec((1, 1, H), lambda n: (l, 0, 0)),
            pl.BlockSpec((1, 1, H), lambda n: (l, 0, 0)),
        ],
        out_specs=pl.BlockSpec((BM, H), lambda n: (n, 0)),
    )(S, xr_flat, h_prev, conv_b, ln_g, ln_b)


def _mlp(h, hW1, hb1, hW2, hb2):
    def body(h_ref, w1_ref, b1_ref, w2_ref, b2_ref, out_ref):
        h2 = jnp.dot(h_ref[...], w1_ref[...],
                     preferred_element_type=jnp.float32) + b1_ref[...]
        h2 = jnp.maximum(h2, 0.0)
        out_ref[...] = jnp.dot(h2, w2_ref[...],
                               preferred_element_type=jnp.float32) + b2_ref[...]
    return pl.pallas_call(
        body,
        grid=(NBLK,),
        out_shape=jax.ShapeDtypeStruct((NP, OUT), jnp.float32),
        in_specs=[
            pl.BlockSpec((BM, H), lambda n: (n, 0)),
            pl.BlockSpec((H, H), lambda n: (0, 0)),
            pl.BlockSpec((1, H), lambda n: (0, 0)),
            pl.BlockSpec((H, OUT), lambda n: (0, 0)),
            pl.BlockSpec((1, OUT), lambda n: (0, 0)),
        ],
        out_specs=pl.BlockSpec((BM, OUT), lambda n: (n, 0)),
    )(h, hW1, hb1, hW2, hb2)


def _edge_layout(a):
    a = a.reshape(NW, E // NW)
    a = jnp.pad(a, ((0, 0), (0, EPT - E // NW)))
    return a.reshape(EROWS, CHUNK)


def kernel(x, edge_index, edge_type, in_W, in_b, bases, comp, root_w,
           conv_b, ln_g, ln_b, hW1, hb1, hW2, hb2):
    src = edge_index[0]
    dst = edge_index[1]
    et = edge_type
    g2d = _edge_layout(et * NP + src)
    dst2d = _edge_layout(dst)
    key2d = _edge_layout(et * NP + dst)
    valid = jnp.ones((NW, E // NW), jnp.float32)
    valid1d = jnp.pad(valid, ((0, 0), (0, EPT - E // NW))).reshape(EP)

    w1d = _count_weights(key2d, valid1d)

    Waug = _combine_weights(bases, comp, root_w)
    x_pad = jnp.pad(x, ((0, NP - N), (0, 0)))
    h = _input_proj(x_pad, in_W, in_b.reshape(1, H))

    cb3 = conv_b.reshape(L, 1, H)
    g3 = ln_g.reshape(L, 1, H)
    b3 = ln_b.reshape(L, 1, H)
    for l in range(L):
        xr_flat = _xr_all(h, Waug, l)
        S = _aggregate(xr_flat, g2d, dst2d, w1d)
        h = _post(S, xr_flat, h, cb3, g3, b3, l, residual=(l > 0))

    out = _mlp(h, hW1, hb1.reshape(1, H), hW2, hb2.reshape(1, OUT))
    return out[:N]


# speedup vs baseline: 16.6645x; 1.8082x over previous
import functools

import jax
import jax.numpy as jnp
from jax import lax
from jax.experimental import pallas as pl
from jax.experimental.pallas import tpu as pltpu
from jax.experimental.pallas import tpu_sc as plsc

N = 10000
E = 160000
D = 128
H = 128
OUT = 64
R = 10
NB = 4
L = 3

NP = 10240
R1 = R + 1
CNTP = R * NP
NC = 2
NS = 16
NW = NC * NS
EPT = 5120
CHUNK = 128
NCH = EPT // CHUNK
EP = NW * EPT
EROWS = EP // CHUNK
CROWS = EP // NS // CHUNK
NROWS_W = NP // NS // CHUNK
CPT = CNTP // NS

BM = 256
NBLK = NP // BM

_MESH = plsc.VectorSubcoreMesh(
    core_axis_name="c", subcore_axis_name="s", num_cores=NC, num_subcores=NS)


def _zero_rows(ref, nrows):
    def body(i, _):
        for q in range(CHUNK // 16):
            ref[i, pl.ds(q * 16, 16)] = jnp.zeros((16,), jnp.float32)
        return 0
    lax.fori_loop(0, nrows, body, 0, unroll=False)


@functools.partial(
    pl.kernel,
    out_type=jax.ShapeDtypeStruct((EP,), jnp.float32),
    mesh=_MESH,
    scratch_types=[
        pltpu.VMEM((CROWS, CHUNK), jnp.int32),
        pltpu.VMEM((EP // NS,), jnp.float32),
        pltpu.VMEM((NCH, CHUNK), jnp.int32),
        pltpu.VMEM((EPT,), jnp.float32),
        pltpu.VMEM((EPT,), jnp.float32),
        pltpu.VMEM((EPT,), jnp.float32),
        pltpu.VMEM((CPT,), jnp.float32),
        pltpu.VMEM_SHARED((CNTP,), jnp.float32),
    ],
)
def _count_weights(key_hbm, valid_hbm, w_hbm,
                   key_a, valid_a, key_b, valid_b, cnt_g, w_v, zbuf, cnt_sh):
    c = lax.axis_index("c")
    s = lax.axis_index("s")
    wid = c * NS + s
    def zbody(i, _):
        zbuf[pl.ds(i * 16, 16)] = jnp.zeros((16,), jnp.float32)
        return 0
    lax.fori_loop(0, CPT // 16, zbody, 0, unroll=False)
    pltpu.sync_copy(zbuf, cnt_sh.at[pl.ds(s * CPT, CPT)])
    plsc.subcore_barrier()
    pltpu.sync_copy(key_hbm.at[pl.ds(s * CROWS, CROWS)], key_a)
    pltpu.sync_copy(valid_hbm.at[pl.ds(s * (EP // NS), EP // NS)], valid_a)

    def cbody(j, _):
        pltpu.sync_copy(valid_a.at[pl.ds(j * CHUNK, CHUNK)],
                        cnt_sh.at[key_a.at[j]], add=True)
        return 0
    lax.fori_loop(0, CROWS, cbody, 0, unroll=False)
    plsc.subcore_barrier()
    pltpu.sync_copy(key_hbm.at[pl.ds(wid * NCH, NCH)], key_b)
    pltpu.sync_copy(valid_hbm.at[pl.ds(wid * EPT, EPT)], valid_b)

    def gbody(j, _):
        pltpu.sync_copy(cnt_sh.at[key_b.at[j]],
                        cnt_g.at[pl.ds(j * CHUNK, CHUNK)])
        return 0
    lax.fori_loop(0, NCH, gbody, 0, unroll=False)

    def wbody(i, _):
        sl = pl.ds(i * 16, 16)
        w_v[sl] = valid_b[sl] / jnp.maximum(cnt_g[sl], 1.0)
        return 0
    lax.fori_loop(0, EPT // 16, wbody, 0, unroll=False)
    pltpu.sync_copy(w_v, w_hbm.at[pl.ds(wid * EPT, EPT)])


@functools.partial(
    pl.kernel,
    out_type=jax.ShapeDtypeStruct((NC, NP, H), jnp.float32),
    mesh=_MESH,
    scratch_types=[
        pltpu.VMEM((NCH, CHUNK), jnp.int32),
        pltpu.VMEM((NCH, CHUNK), jnp.int32),
        pltpu.VMEM((EPT,), jnp.float32),
        pltpu.VMEM((CHUNK, H), jnp.float32),
        pltpu.VMEM((CHUNK, H), jnp.float32),
        pltpu.VMEM_SHARED((NP, H), jnp.float32),
        pltpu.SemaphoreType.DMA,
        pltpu.SemaphoreType.DMA,
    ],
    compiler_params=pltpu.CompilerParams(needs_layout_passes=False),
)
def _aggregate(xr_hbm, g_hbm, dst_hbm, w_hbm, out_hbm,
               g_v, dst_v, w_v, rows0, rows1, acc_sh, sem, sem_s):
    c = lax.axis_index("c")
    s = lax.axis_index("s")
    wid = c * NS + s
    _zero_rows(rows0, CHUNK)
    for t in range(NROWS_W):
        pltpu.sync_copy(rows0, acc_sh.at[pl.ds((s * NROWS_W + t) * CHUNK, CHUNK)])
    plsc.subcore_barrier()
    pltpu.sync_copy(g_hbm.at[pl.ds(wid * NCH, NCH)], g_v)
    pltpu.sync_copy(dst_hbm.at[pl.ds(wid * NCH, NCH)], dst_v)
    pltpu.sync_copy(w_hbm.at[pl.ds(wid * EPT, EPT)], w_v)

    bufs = (rows0, rows1)
    pltpu.async_copy(xr_hbm.at[g_v.at[0]], rows0, sem)

    def scale(j, buf):
        jbase = j * CHUNK

        def grp_body(t, _):
            wg = w_v[pl.ds(jbase + t * 16, 16)]
            for r in range(16):
                ws = wg[r]
                for q in range(H // 16):
                    sl = pl.ds(q * 16, 16)
                    buf[t * 16 + r, sl] = buf[t * 16 + r, sl] * ws
            return 0
        lax.fori_loop(0, CHUNK // 16, grp_body, 0, unroll=False)

    def pair_body(j2, _):
        for b in range(2):
            j = j2 * 2 + b
            cur, nxt = bufs[b], bufs[1 - b]

            @pl.when(j >= 1)
            def _():
                pltpu.make_async_copy(
                    nxt, acc_sh.at[dst_v.at[j - 1]], sem_s).wait()

            @pl.when(j < NCH - 1)
            def _():
                pltpu.async_copy(xr_hbm.at[g_v.at[j + 1]], nxt, sem)
            pltpu.make_async_copy(xr_hbm.at[g_v.at[j]], cur, sem).wait()
            scale(j, cur)
            pltpu.async_copy(cur, acc_sh.at[dst_v.at[j]], sem_s, add=True)
        return 0
    lax.fori_loop(0, NCH // 2, pair_body, 0, unroll=False)
    pltpu.make_async_copy(
        bufs[1], acc_sh.at[dst_v.at[NCH - 1]], sem_s).wait()
    plsc.subcore_barrier()
    for t in range(NROWS_W):
        rs = pl.ds((s * NROWS_W + t) * CHUNK, CHUNK)
        pltpu.sync_copy(acc_sh.at[rs], rows0)
        pltpu.sync_copy(rows0, out_hbm.at[c, rs])


def _combine_weights(bases, comp, root_w):
    def body(comp_ref, bases_ref, root_ref, out_ref):
        for l in range(L):
            for r in range(R):
                acc = comp_ref[l, r, 0] * bases_ref[l, 0]
                for b in range(1, NB):
                    acc = acc + comp_ref[l, r, b] * bases_ref[l, b]
                out_ref[l, r] = acc
            out_ref[l, R] = root_ref[l]
    return pl.pallas_call(
        body,
        out_shape=jax.ShapeDtypeStruct((L, R1, H, H), jnp.float32),
        in_specs=[
            pl.BlockSpec(memory_space=pltpu.SMEM),
            pl.BlockSpec(memory_space=pltpu.VMEM),
            pl.BlockSpec(memory_space=pltpu.VMEM),
        ],
        out_specs=pl.BlockSpec(memory_space=pltpu.VMEM),
    )(comp, bases, root_w)


def _in_xr(x_pad, in_W, in_b, Waug):
    def body(x_ref, w_ref, b_ref, wa_ref, h_ref, xr_ref):
        h0 = jnp.dot(x_ref[...], w_ref[...],
                     preferred_element_type=jnp.float32) + b_ref[...]
        h_ref[...] = h0
        for r in range(R1):
            xr_ref[r] = jnp.dot(h0, wa_ref[0, r],
                                preferred_element_type=jnp.float32)
    return pl.pallas_call(
        body,
        grid=(NBLK,),
        out_shape=(jax.ShapeDtypeStruct((NP, H), jnp.float32),
                   jax.ShapeDtypeStruct((R1, NP, H), jnp.float32)),
        in_specs=[
            pl.BlockSpec((BM, D), lambda n: (n, 0)),
            pl.BlockSpec((D, H), lambda n: (0, 0)),
            pl.BlockSpec((1, H), lambda n: (0, 0)),
            pl.BlockSpec((1, R1, H, H), lambda n: (0, 0, 0, 0)),
        ],
        out_specs=(pl.BlockSpec((BM, H), lambda n: (n, 0)),
                   pl.BlockSpec((R1, BM, H), lambda n: (0, n, 0))),
    )(x_pad, in_W, in_b, Waug)


def _post_ln(s_ref, xr_ref, h_ref, cb_ref, g_ref, b_ref, residual):
    t = xr_ref[R] + s_ref[0] + s_ref[1] + cb_ref[0]
    m = jnp.mean(t, axis=1, keepdims=True)
    d = t - m
    v = jnp.mean(d * d, axis=1, keepdims=True)
    hn = d * lax.rsqrt(v + 1e-5) * g_ref[0] + b_ref[0]
    hn = jnp.maximum(hn, 0.0)
    if residual:
        hn = hn + h_ref[...]
    return hn


def _post_xr(S, xr_prev, h_prev, conv_b, ln_g, ln_b, Waug, l):
    def body(s_ref, xr_ref, h_ref, cb_ref, g_ref, b_ref, wa_ref,
             hout_ref, xrout_ref):
        hn = _post_ln(s_ref, xr_ref, h_ref, cb_ref, g_ref, b_ref,
                      residual=(l > 0))
        hout_ref[...] = hn
        for r in range(R1):
            xrout_ref[r] = jnp.dot(hn, wa_ref[0, r],
                                   preferred_element_type=jnp.float32)
    return pl.pallas_call(
        body,
        grid=(NBLK,),
        out_shape=(jax.ShapeDtypeStruct((NP, H), jnp.float32),
                   jax.ShapeDtypeStruct((R1, NP, H), jnp.float32)),
        in_specs=[
            pl.BlockSpec((NC, BM, H), lambda n: (0, n, 0)),
            pl.BlockSpec((R1, BM, H), lambda n: (0, n, 0)),
            pl.BlockSpec((BM, H), lambda n: (n, 0)),
            pl.BlockSpec((1, 1, H), lambda n: (l, 0, 0)),
            pl.BlockSpec((1, 1, H), lambda n: (l, 0, 0)),
            pl.BlockSpec((1, 1, H), lambda n: (l, 0, 0)),
            pl.BlockSpec((1, R1, H, H), lambda n: (l + 1, 0, 0, 0)),
        ],
        out_specs=(pl.BlockSpec((BM, H), lambda n: (n, 0)),
                   pl.BlockSpec((R1, BM, H), lambda n: (0, n, 0))),
    )(S, xr_prev, h_prev, conv_b, ln_g, ln_b, Waug)


def _post_mlp(S, xr_prev, h_prev, conv_b, ln_g, ln_b, hW1, hb1, hW2, hb2, l):
    def body(s_ref, xr_ref, h_ref, cb_ref, g_ref, b_ref,
             w1_ref, b1_ref, w2_ref, b2_ref, out_ref):
        hn = _post_ln(s_ref, xr_ref, h_ref, cb_ref, g_ref, b_ref,
                      residual=True)
        h2 = jnp.dot(hn, w1_ref[...],
                     preferred_element_type=jnp.float32) + b1_ref[...]
        h2 = jnp.maximum(h2, 0.0)
        out_ref[...] = jnp.dot(h2, w2_ref[...],
                               preferred_element_type=jnp.float32) + b2_ref[...]
    return pl.pallas_call(
        body,
        grid=(NBLK,),
        out_shape=jax.ShapeDtypeStruct((NP, OUT), jnp.float32),
        in_specs=[
            pl.BlockSpec((NC, BM, H), lambda n: (0, n, 0)),
            pl.BlockSpec((R1, BM, H), lambda n: (0, n, 0)),
            pl.BlockSpec((BM, H), lambda n: (n, 0)),
            pl.BlockSpec((1, 1, H), lambda n: (l, 0, 0)),
            pl.BlockSpec((1, 1, H), lambda n: (l, 0, 0)),
            pl.BlockSpec((1, 1, H), lambda n: (l, 0, 0)),
            pl.BlockSpec((H, H), lambda n: (0, 0)),
            pl.BlockSpec((1, H), lambda n: (0, 0)),
            pl.BlockSpec((H, OUT), lambda n: (0, 0)),
            pl.BlockSpec((1, OUT), lambda n: (0, 0)),
        ],
        out_specs=pl.BlockSpec((BM, OUT), lambda n: (n, 0)),
    )(S, xr_prev, h_prev, conv_b, ln_g, ln_b, hW1, hb1, hW2, hb2)


def _edge_layout(a):
    a = a.reshape(NW, E // NW)
    a = jnp.pad(a, ((0, 0), (0, EPT - E // NW)))
    return a.reshape(EROWS, CHUNK)


def kernel(x, edge_index, edge_type, in_W, in_b, bases, comp, root_w,
           conv_b, ln_g, ln_b, hW1, hb1, hW2, hb2):
    src = edge_index[0]
    dst = edge_index[1]
    et = edge_type
    g2d = _edge_layout(et * NP + src)
    dst2d = _edge_layout(dst)
    key2d = _edge_layout(et * NP + dst)
    valid = jnp.ones((NW, E // NW), jnp.float32)
    valid1d = jnp.pad(valid, ((0, 0), (0, EPT - E // NW))).reshape(EP)

    w1d = _count_weights(key2d, valid1d)

    Waug = _combine_weights(bases, comp, root_w)
    x_pad = jnp.pad(x, ((0, NP - N), (0, 0)))
    h, xr = _in_xr(x_pad, in_W, in_b.reshape(1, H), Waug)

    cb3 = conv_b.reshape(L, 1, H)
    g3 = ln_g.reshape(L, 1, H)
    b3 = ln_b.reshape(L, 1, H)
    out = None
    for l in range(L):
        S = _aggregate(xr.reshape(R1 * NP, H), g2d, dst2d, w1d)
        if l < L - 1:
            h, xr = _post_xr(S, xr, h, cb3, g3, b3, Waug, l)
        else:
            out = _post_mlp(S, xr, h, cb3, g3, b3, hW1,
                            hb1.reshape(1, H), hW2, hb2.reshape(1, OUT), l)
    return out[:N]


# speedup vs baseline: 17.2863x; 1.0373x over previous
import functools

import jax
import jax.numpy as jnp
import numpy as np
from jax import lax
from jax.experimental import pallas as pl
from jax.experimental.pallas import tpu as pltpu
from jax.experimental.pallas import tpu_sc as plsc

N = 10000
E = 160000
D = 128
H = 128
OUT = 64
R = 10
NB = 4
L = 3

NP = 10240
R1 = R + 1
CNTP = R * NP
NC = 2
NS = 16
NW = NC * NS
EPT = 5120
CHUNK = 128
NCH = EPT // CHUNK
EP = NW * EPT
EROWS = EP // CHUNK
CROWS = EP // NS // CHUNK
NROWS_W = NP // NS // CHUNK
CPT = CNTP // NS

BM = 256
NBLK = NP // BM
WPR = H // 2

_PERM_NAT = np.zeros(H, np.int64)
for _q in range(H // 32):
    _b = 32 * _q
    for _k in range(16):
        _PERM_NAT[_b + 2 * _k] = _b + _k
        _PERM_NAT[_b + 2 * _k + 1] = _b + 16 + _k
_PMAT = np.zeros((H, H), np.float32)
_PMAT[_PERM_NAT, np.arange(H)] = 1.0

_MESH = plsc.VectorSubcoreMesh(
    core_axis_name="c", subcore_axis_name="s", num_cores=NC, num_subcores=NS)


def _zero_rows(ref, nrows):
    def body(i, _):
        for q in range(CHUNK // 16):
            ref[i, pl.ds(q * 16, 16)] = jnp.zeros((16,), jnp.float32)
        return 0
    lax.fori_loop(0, nrows, body, 0, unroll=False)


@functools.partial(
    pl.kernel,
    out_type=jax.ShapeDtypeStruct((EP,), jnp.float32),
    mesh=_MESH,
    scratch_types=[
        pltpu.VMEM((CROWS, CHUNK), jnp.int32),
        pltpu.VMEM((EP // NS,), jnp.float32),
        pltpu.VMEM((NCH, CHUNK), jnp.int32),
        pltpu.VMEM((EPT,), jnp.float32),
        pltpu.VMEM((EPT,), jnp.float32),
        pltpu.VMEM((EPT,), jnp.float32),
        pltpu.VMEM((CPT,), jnp.float32),
        pltpu.VMEM_SHARED((CNTP,), jnp.float32),
    ],
)
def _count_weights(key_hbm, valid_hbm, w_hbm,
                   key_a, valid_a, key_b, valid_b, cnt_g, w_v, zbuf, cnt_sh):
    c = lax.axis_index("c")
    s = lax.axis_index("s")
    wid = c * NS + s
    def zbody(i, _):
        zbuf[pl.ds(i * 16, 16)] = jnp.zeros((16,), jnp.float32)
        return 0
    lax.fori_loop(0, CPT // 16, zbody, 0, unroll=False)
    pltpu.sync_copy(zbuf, cnt_sh.at[pl.ds(s * CPT, CPT)])
    plsc.subcore_barrier()
    pltpu.sync_copy(key_hbm.at[pl.ds(s * CROWS, CROWS)], key_a)
    pltpu.sync_copy(valid_hbm.at[pl.ds(s * (EP // NS), EP // NS)], valid_a)

    def cbody(j, _):
        pltpu.sync_copy(valid_a.at[pl.ds(j * CHUNK, CHUNK)],
                        cnt_sh.at[key_a.at[j]], add=True)
        return 0
    lax.fori_loop(0, CROWS, cbody, 0, unroll=False)
    plsc.subcore_barrier()
    pltpu.sync_copy(key_hbm.at[pl.ds(wid * NCH, NCH)], key_b)
    pltpu.sync_copy(valid_hbm.at[pl.ds(wid * EPT, EPT)], valid_b)

    def gbody(j, _):
        pltpu.sync_copy(cnt_sh.at[key_b.at[j]],
                        cnt_g.at[pl.ds(j * CHUNK, CHUNK)])
        return 0
    lax.fori_loop(0, NCH, gbody, 0, unroll=False)

    def wbody(i, _):
        sl = pl.ds(i * 16, 16)
        w_v[sl] = valid_b[sl] / jnp.maximum(cnt_g[sl], 1.0)
        return 0
    lax.fori_loop(0, EPT // 16, wbody, 0, unroll=False)
    pltpu.sync_copy(w_v, w_hbm.at[pl.ds(wid * EPT, EPT)])


@functools.partial(
    pl.kernel,
    out_type=jax.ShapeDtypeStruct((NC, NP, H), jnp.float32),
    mesh=_MESH,
    scratch_types=[
        pltpu.VMEM((NCH, CHUNK), jnp.int32),
        pltpu.VMEM((NCH, CHUNK), jnp.int32),
        pltpu.VMEM((EPT,), jnp.float32),
        pltpu.VMEM((CHUNK, H), jnp.float32),
        pltpu.VMEM((CHUNK, H), jnp.float32),
        pltpu.VMEM_SHARED((NP, H), jnp.float32),
        pltpu.SemaphoreType.DMA,
        pltpu.SemaphoreType.DMA,
    ],
    compiler_params=pltpu.CompilerParams(needs_layout_passes=False),
)
def _aggregate(xr_hbm, g_hbm, dst_hbm, w_hbm, out_hbm,
               g_v, dst_v, w_v, in0, in1, acc_sh, sem, sem_s):
    c = lax.axis_index("c")
    s = lax.axis_index("s")
    wid = c * NS + s
    _zero_rows(in0, CHUNK)
    for t in range(NROWS_W):
        pltpu.sync_copy(in0, acc_sh.at[pl.ds((s * NROWS_W + t) * CHUNK, CHUNK)])
    plsc.subcore_barrier()
    pltpu.sync_copy(g_hbm.at[pl.ds(wid * NCH, NCH)], g_v)
    pltpu.sync_copy(dst_hbm.at[pl.ds(wid * NCH, NCH)], dst_v)
    pltpu.sync_copy(w_hbm.at[pl.ds(wid * EPT, EPT)], w_v)

    bufs = (in0, in1)
    HC = CHUNK // 2

    def start_gather(j, buf):
        pltpu.async_copy(xr_hbm.at[g_v.at[j, pl.ds(0, HC)]],
                         buf.at[pl.ds(0, HC)], sem)
        pltpu.async_copy(xr_hbm.at[g_v.at[j, pl.ds(HC, HC)]],
                         buf.at[pl.ds(HC, HC)], sem)

    def wait_gather(j, buf):
        pltpu.make_async_copy(xr_hbm.at[g_v.at[j, pl.ds(0, HC)]],
                              buf.at[pl.ds(0, HC)], sem).wait()
        pltpu.make_async_copy(xr_hbm.at[g_v.at[j, pl.ds(HC, HC)]],
                              buf.at[pl.ds(HC, HC)], sem).wait()

    start_gather(0, in0)

    def scale(j, buf):
        jbase = j * CHUNK

        def grp_body(t, _):
            wg = w_v[pl.ds(jbase + t * 16, 16)]
            for r in range(16):
                ws = wg[r]
                row = t * 16 + r
                for q in range(H // 16):
                    sl = pl.ds(q * 16, 16)
                    buf[row, sl] = buf[row, sl] * ws
            return 0
        lax.fori_loop(0, CHUNK // 16, grp_body, 0, unroll=False)

    def pair_body(j2, _):
        for b in range(2):
            j = j2 * 2 + b
            cur, nxt = bufs[b], bufs[1 - b]

            @pl.when(j >= 1)
            def _():
                pltpu.make_async_copy(
                    nxt, acc_sh.at[dst_v.at[j - 1]], sem_s).wait()

            @pl.when(j < NCH - 1)
            def _():
                start_gather(j + 1, nxt)
            wait_gather(j, cur)
            scale(j, cur)
            pltpu.async_copy(cur, acc_sh.at[dst_v.at[j]], sem_s, add=True)
        return 0
    lax.fori_loop(0, NCH // 2, pair_body, 0, unroll=False)
    pltpu.make_async_copy(
        bufs[1], acc_sh.at[dst_v.at[NCH - 1]], sem_s).wait()
    plsc.subcore_barrier()
    for t in range(NROWS_W):
        rs = pl.ds((s * NROWS_W + t) * CHUNK, CHUNK)
        pltpu.sync_copy(acc_sh.at[rs], in0)
        pltpu.sync_copy(in0, out_hbm.at[c, rs])


def _combine_weights(bases, comp, root_w, pmat):
    def body(comp_ref, bases_ref, root_ref, p_ref, out_ref):
        del p_ref
        for l in range(L):
            for r in range(R):
                acc = comp_ref[l, r, 0] * bases_ref[l, 0]
                for b in range(1, NB):
                    acc = acc + comp_ref[l, r, b] * bases_ref[l, b]
                out_ref[l, r] = acc
            out_ref[l, R] = root_ref[l]
    return pl.pallas_call(
        body,
        out_shape=jax.ShapeDtypeStruct((L, R1, H, H), jnp.float32),
        in_specs=[
            pl.BlockSpec(memory_space=pltpu.SMEM),
            pl.BlockSpec(memory_space=pltpu.VMEM),
            pl.BlockSpec(memory_space=pltpu.VMEM),
            pl.BlockSpec(memory_space=pltpu.VMEM),
        ],
        out_specs=pl.BlockSpec(memory_space=pltpu.VMEM),
    )(comp, bases, root_w, pmat)


def _slabs(h0, wa_ref, xrsc_ref, xrroot_ref):
    for r in range(R):
        xrsc_ref[r] = jnp.dot(h0, wa_ref[0, r],
                              preferred_element_type=jnp.float32)
    xrroot_ref[...] = jnp.dot(h0, wa_ref[0, R],
                              preferred_element_type=jnp.float32)


def _in_xr(x_pad, in_W, in_b, Waug):
    def body(x_ref, w_ref, b_ref, wa_ref, h_ref, xrsc_ref, xrroot_ref):
        h0 = jnp.dot(x_ref[...], w_ref[...],
                     preferred_element_type=jnp.float32) + b_ref[...]
        h_ref[...] = h0
        _slabs(h0, wa_ref, xrsc_ref, xrroot_ref)
    return pl.pallas_call(
        body,
        grid=(NBLK,),
        out_shape=(jax.ShapeDtypeStruct((NP, H), jnp.float32),
                   jax.ShapeDtypeStruct((R, NP, H), jnp.float32),
                   jax.ShapeDtypeStruct((NP, H), jnp.float32)),
        in_specs=[
            pl.BlockSpec((BM, D), lambda n: (n, 0)),
            pl.BlockSpec((D, H), lambda n: (0, 0)),
            pl.BlockSpec((1, H), lambda n: (0, 0)),
            pl.BlockSpec((1, R1, H, H), lambda n: (0, 0, 0, 0)),
        ],
        out_specs=(pl.BlockSpec((BM, H), lambda n: (n, 0)),
                   pl.BlockSpec((R, BM, H), lambda n: (0, n, 0)),
                   pl.BlockSpec((BM, H), lambda n: (n, 0))),
    )(x_pad, in_W, in_b, Waug)


def _post_ln(s_ref, xr_ref, h_ref, cb_ref, g_ref, b_ref, residual):
    t = xr_ref[...] + s_ref[0] + s_ref[1] + cb_ref[0]
    m = jnp.mean(t, axis=1, keepdims=True)
    d = t - m
    v = jnp.mean(d * d, axis=1, keepdims=True)
    hn = d * lax.rsqrt(v + 1e-5) * g_ref[0] + b_ref[0]
    hn = jnp.maximum(hn, 0.0)
    if residual:
        hn = hn + h_ref[...]
    return hn


def _post_xr(S, xr_root, h_prev, conv_b, ln_g, ln_b, Waug, l):
    def body(s_ref, xr_ref, h_ref, cb_ref, g_ref, b_ref, wa_ref,
             hout_ref, xrsc_ref, xrroot_ref):
        hn = _post_ln(s_ref, xr_ref, h_ref, cb_ref, g_ref, b_ref,
                      residual=(l > 0))
        hout_ref[...] = hn
        _slabs(hn, wa_ref, xrsc_ref, xrroot_ref)
    return pl.pallas_call(
        body,
        grid=(NBLK,),
        out_shape=(jax.ShapeDtypeStruct((NP, H), jnp.float32),
                   jax.ShapeDtypeStruct((R, NP, H), jnp.float32),
                   jax.ShapeDtypeStruct((NP, H), jnp.float32)),
        in_specs=[
            pl.BlockSpec((NC, BM, H), lambda n: (0, n, 0)),
            pl.BlockSpec((BM, H), lambda n: (n, 0)),
            pl.BlockSpec((BM, H), lambda n: (n, 0)),
            pl.BlockSpec((1, 1, H), lambda n: (l, 0, 0)),
            pl.BlockSpec((1, 1, H), lambda n: (l, 0, 0)),
            pl.BlockSpec((1, 1, H), lambda n: (l, 0, 0)),
            pl.BlockSpec((1, R1, H, H), lambda n: (l + 1, 0, 0, 0)),
        ],
        out_specs=(pl.BlockSpec((BM, H), lambda n: (n, 0)),
                   pl.BlockSpec((R, BM, H), lambda n: (0, n, 0)),
                   pl.BlockSpec((BM, H), lambda n: (n, 0))),
    )(S, xr_root, h_prev, conv_b, ln_g, ln_b, Waug)


def _post_mlp(S, xr_root, h_prev, conv_b, ln_g, ln_b, hW1, hb1, hW2, hb2, l):
    def body(s_ref, xr_ref, h_ref, cb_ref, g_ref, b_ref,
             w1_ref, b1_ref, w2_ref, b2_ref, out_ref):
        hn = _post_ln(s_ref, xr_ref, h_ref, cb_ref, g_ref, b_ref,
                      residual=True)
        h2 = jnp.dot(hn, w1_ref[...],
                     preferred_element_type=jnp.float32) + b1_ref[...]
        h2 = jnp.maximum(h2, 0.0)
        out_ref[...] = jnp.dot(h2, w2_ref[...],
                               preferred_element_type=jnp.float32) + b2_ref[...]
    return pl.pallas_call(
        body,
        grid=(NBLK,),
        out_shape=jax.ShapeDtypeStruct((NP, OUT), jnp.float32),
        in_specs=[
            pl.BlockSpec((NC, BM, H), lambda n: (0, n, 0)),
            pl.BlockSpec((BM, H), lambda n: (n, 0)),
            pl.BlockSpec((BM, H), lambda n: (n, 0)),
            pl.BlockSpec((1, 1, H), lambda n: (l, 0, 0)),
            pl.BlockSpec((1, 1, H), lambda n: (l, 0, 0)),
            pl.BlockSpec((1, 1, H), lambda n: (l, 0, 0)),
            pl.BlockSpec((H, H), lambda n: (0, 0)),
            pl.BlockSpec((1, H), lambda n: (0, 0)),
            pl.BlockSpec((H, OUT), lambda n: (0, 0)),
            pl.BlockSpec((1, OUT), lambda n: (0, 0)),
        ],
        out_specs=pl.BlockSpec((BM, OUT), lambda n: (n, 0)),
    )(S, xr_root, h_prev, conv_b, ln_g, ln_b, hW1, hb1, hW2, hb2)


def _edge_layout(a):
    a = a.reshape(NW, E // NW)
    a = jnp.pad(a, ((0, 0), (0, EPT - E // NW)))
    return a.reshape(EROWS, CHUNK)


def kernel(x, edge_index, edge_type, in_W, in_b, bases, comp, root_w,
           conv_b, ln_g, ln_b, hW1, hb1, hW2, hb2):
    src = edge_index[0]
    dst = edge_index[1]
    et = edge_type
    g2d = _edge_layout(et * NP + src)
    dst2d = _edge_layout(dst)
    key2d = _edge_layout(et * NP + dst)
    valid = jnp.ones((NW, E // NW), jnp.float32)
    valid1d = jnp.pad(valid, ((0, 0), (0, EPT - E // NW))).reshape(EP)

    w1d = _count_weights(key2d, valid1d)

    Waug = _combine_weights(bases, comp, root_w, jnp.asarray(_PMAT))
    x_pad = jnp.pad(x, ((0, NP - N), (0, 0)))
    h, xr_sc, xr_root = _in_xr(x_pad, in_W, in_b.reshape(1, H), Waug)

    cb3 = conv_b.reshape(L, 1, H)
    g3 = ln_g.reshape(L, 1, H)
    b3 = ln_b.reshape(L, 1, H)
    out = None
    for l in range(L):
        S = _aggregate(xr_sc.reshape(R * NP, H), g2d, dst2d, w1d)
        if l < L - 1:
            h, xr_sc, xr_root = _post_xr(S, xr_root, h, cb3, g3, b3, Waug, l)
        else:
            out = _post_mlp(S, xr_root, h, cb3, g3, b3, hW1,
                            hb1.reshape(1, H), hW2, hb2.reshape(1, OUT), l)
    return out[:N]


# speedup vs baseline: 17.2901x; 1.0002x over previous
import functools

import jax
import jax.numpy as jnp
import numpy as np
from jax import lax
from jax.experimental import pallas as pl
from jax.experimental.pallas import tpu as pltpu
from jax.experimental.pallas import tpu_sc as plsc

N = 10000
E = 160000
D = 128
H = 128
OUT = 64
R = 10
NB = 4
L = 3

NP = 10240
R1 = R + 1
CNTP = R * NP
NC = 2
NS = 16
NW = NC * NS
EPT = 5120
CHUNK = 128
NCH = EPT // CHUNK
EP = NW * EPT
EROWS = EP // CHUNK
CROWS = EP // NS // CHUNK
NROWS_W = NP // NS // CHUNK
CPT = CNTP // NS

BM = 256
NBLK = NP // BM
WPR = H // 2

_PERM_NAT = np.zeros(H, np.int64)
for _q in range(H // 32):
    _b = 32 * _q
    for _k in range(16):
        _PERM_NAT[_b + 2 * _k] = _b + _k
        _PERM_NAT[_b + 2 * _k + 1] = _b + 16 + _k
_PMAT = np.zeros((H, H), np.float32)
_PMAT[_PERM_NAT, np.arange(H)] = 1.0

_MESH = plsc.VectorSubcoreMesh(
    core_axis_name="c", subcore_axis_name="s", num_cores=NC, num_subcores=NS)


def _zero_rows(ref, nrows):
    def body(i, _):
        for q in range(CHUNK // 16):
            ref[i, pl.ds(q * 16, 16)] = jnp.zeros((16,), jnp.float32)
        return 0
    lax.fori_loop(0, nrows, body, 0, unroll=False)


@functools.partial(
    pl.kernel,
    out_type=jax.ShapeDtypeStruct((EP,), jnp.float32),
    mesh=_MESH,
    scratch_types=[
        pltpu.VMEM((CROWS, CHUNK), jnp.int32),
        pltpu.VMEM((EP // NS,), jnp.float32),
        pltpu.VMEM((NCH, CHUNK), jnp.int32),
        pltpu.VMEM((EPT,), jnp.float32),
        pltpu.VMEM((EPT,), jnp.float32),
        pltpu.VMEM((EPT,), jnp.float32),
        pltpu.VMEM((CPT,), jnp.float32),
        pltpu.VMEM_SHARED((CNTP,), jnp.float32),
    ],
)
def _count_weights(key_hbm, valid_hbm, w_hbm,
                   key_a, valid_a, key_b, valid_b, cnt_g, w_v, zbuf, cnt_sh):
    c = lax.axis_index("c")
    s = lax.axis_index("s")
    wid = c * NS + s
    def zbody(i, _):
        zbuf[pl.ds(i * 16, 16)] = jnp.zeros((16,), jnp.float32)
        return 0
    lax.fori_loop(0, CPT // 16, zbody, 0, unroll=False)
    pltpu.sync_copy(zbuf, cnt_sh.at[pl.ds(s * CPT, CPT)])
    plsc.subcore_barrier()
    pltpu.sync_copy(key_hbm.at[pl.ds(s * CROWS, CROWS)], key_a)
    pltpu.sync_copy(valid_hbm.at[pl.ds(s * (EP // NS), EP // NS)], valid_a)

    def cbody(j, _):
        pltpu.sync_copy(valid_a.at[pl.ds(j * CHUNK, CHUNK)],
                        cnt_sh.at[key_a.at[j]], add=True)
        return 0
    lax.fori_loop(0, CROWS, cbody, 0, unroll=False)
    plsc.subcore_barrier()
    pltpu.sync_copy(key_hbm.at[pl.ds(wid * NCH, NCH)], key_b)
    pltpu.sync_copy(valid_hbm.at[pl.ds(wid * EPT, EPT)], valid_b)

    def gbody(j, _):
        pltpu.sync_copy(cnt_sh.at[key_b.at[j]],
                        cnt_g.at[pl.ds(j * CHUNK, CHUNK)])
        return 0
    lax.fori_loop(0, NCH, gbody, 0, unroll=False)

    def wbody(i, _):
        sl = pl.ds(i * 16, 16)
        w_v[sl] = valid_b[sl] / jnp.maximum(cnt_g[sl], 1.0)
        return 0
    lax.fori_loop(0, EPT // 16, wbody, 0, unroll=False)
    pltpu.sync_copy(w_v, w_hbm.at[pl.ds(wid * EPT, EPT)])


@functools.partial(
    pl.kernel,
    out_type=jax.ShapeDtypeStruct((NC, NP, H), jnp.float32),
    mesh=_MESH,
    scratch_types=[
        pltpu.VMEM((NCH, CHUNK), jnp.int32),
        pltpu.VMEM((NCH, CHUNK), jnp.int32),
        pltpu.VMEM((EPT,), jnp.float32),
        pltpu.VMEM((CHUNK, H), jnp.float32),
        pltpu.VMEM((CHUNK, H), jnp.float32),
        pltpu.VMEM_SHARED((NP, H), jnp.float32),
        pltpu.SemaphoreType.DMA,
        pltpu.SemaphoreType.DMA,
    ],
    compiler_params=pltpu.CompilerParams(needs_layout_passes=False),
)
def _aggregate(xr_hbm, g_hbm, dst_hbm, w_hbm, out_hbm,
               g_v, dst_v, w_v, in0, in1, acc_sh, sem, sem_s):
    c = lax.axis_index("c")
    s = lax.axis_index("s")
    wid = c * NS + s
    _zero_rows(in0, CHUNK)
    for t in range(NROWS_W):
        pltpu.sync_copy(in0, acc_sh.at[pl.ds((s * NROWS_W + t) * CHUNK, CHUNK)])
    plsc.subcore_barrier()
    pltpu.sync_copy(g_hbm.at[pl.ds(wid * NCH, NCH)], g_v)
    pltpu.sync_copy(dst_hbm.at[pl.ds(wid * NCH, NCH)], dst_v)
    pltpu.sync_copy(w_hbm.at[pl.ds(wid * EPT, EPT)], w_v)

    bufs = (in0, in1)
    NSPLIT = 4
    HC = CHUNK // NSPLIT

    def start_gather(j, buf):
        for p in range(NSPLIT):
            pltpu.async_copy(xr_hbm.at[g_v.at[j, pl.ds(p * HC, HC)]],
                             buf.at[pl.ds(p * HC, HC)], sem)

    def wait_gather(j, buf):
        for p in range(NSPLIT):
            pltpu.make_async_copy(xr_hbm.at[g_v.at[j, pl.ds(p * HC, HC)]],
                                  buf.at[pl.ds(p * HC, HC)], sem).wait()

    start_gather(0, in0)

    def scale(j, buf):
        jbase = j * CHUNK

        def grp_body(t, _):
            wg = w_v[pl.ds(jbase + t * 16, 16)]
            for r in range(16):
                ws = wg[r]
                row = t * 16 + r
                for q in range(H // 16):
                    sl = pl.ds(q * 16, 16)
                    buf[row, sl] = buf[row, sl] * ws
            return 0
        lax.fori_loop(0, CHUNK // 16, grp_body, 0, unroll=False)

    def pair_body(j2, _):
        for b in range(2):
            j = j2 * 2 + b
            cur, nxt = bufs[b], bufs[1 - b]

            @pl.when(j >= 1)
            def _():
                pltpu.make_async_copy(
                    nxt, acc_sh.at[dst_v.at[j - 1]], sem_s).wait()

            @pl.when(j < NCH - 1)
            def _():
                start_gather(j + 1, nxt)
            wait_gather(j, cur)
            scale(j, cur)
            pltpu.async_copy(cur, acc_sh.at[dst_v.at[j]], sem_s, add=True)
        return 0
    lax.fori_loop(0, NCH // 2, pair_body, 0, unroll=False)
    pltpu.make_async_copy(
        bufs[1], acc_sh.at[dst_v.at[NCH - 1]], sem_s).wait()
    plsc.subcore_barrier()
    for t in range(NROWS_W):
        rs = pl.ds((s * NROWS_W + t) * CHUNK, CHUNK)
        pltpu.sync_copy(acc_sh.at[rs], in0)
        pltpu.sync_copy(in0, out_hbm.at[c, rs])


def _combine_weights(bases, comp, root_w, pmat):
    def body(comp_ref, bases_ref, root_ref, p_ref, out_ref):
        del p_ref
        for l in range(L):
            for r in range(R):
                acc = comp_ref[l, r, 0] * bases_ref[l, 0]
                for b in range(1, NB):
                    acc = acc + comp_ref[l, r, b] * bases_ref[l, b]
                out_ref[l, r] = acc
            out_ref[l, R] = root_ref[l]
    return pl.pallas_call(
        body,
        out_shape=jax.ShapeDtypeStruct((L, R1, H, H), jnp.float32),
        in_specs=[
            pl.BlockSpec(memory_space=pltpu.SMEM),
            pl.BlockSpec(memory_space=pltpu.VMEM),
            pl.BlockSpec(memory_space=pltpu.VMEM),
            pl.BlockSpec(memory_space=pltpu.VMEM),
        ],
        out_specs=pl.BlockSpec(memory_space=pltpu.VMEM),
    )(comp, bases, root_w, pmat)


def _slabs(h0, wa_ref, xrsc_ref, xrroot_ref):
    for r in range(R):
        xrsc_ref[r] = jnp.dot(h0, wa_ref[0, r],
                              preferred_element_type=jnp.float32)
    xrroot_ref[...] = jnp.dot(h0, wa_ref[0, R],
                              preferred_element_type=jnp.float32)


def _in_xr(x_pad, in_W, in_b, Waug):
    def body(x_ref, w_ref, b_ref, wa_ref, h_ref, xrsc_ref, xrroot_ref):
        h0 = jnp.dot(x_ref[...], w_ref[...],
                     preferred_element_type=jnp.float32) + b_ref[...]
        h_ref[...] = h0
        _slabs(h0, wa_ref, xrsc_ref, xrroot_ref)
    return pl.pallas_call(
        body,
        grid=(NBLK,),
        out_shape=(jax.ShapeDtypeStruct((NP, H), jnp.float32),
                   jax.ShapeDtypeStruct((R, NP, H), jnp.float32),
                   jax.ShapeDtypeStruct((NP, H), jnp.float32)),
        in_specs=[
            pl.BlockSpec((BM, D), lambda n: (n, 0)),
            pl.BlockSpec((D, H), lambda n: (0, 0)),
            pl.BlockSpec((1, H), lambda n: (0, 0)),
            pl.BlockSpec((1, R1, H, H), lambda n: (0, 0, 0, 0)),
        ],
        out_specs=(pl.BlockSpec((BM, H), lambda n: (n, 0)),
                   pl.BlockSpec((R, BM, H), lambda n: (0, n, 0)),
                   pl.BlockSpec((BM, H), lambda n: (n, 0))),
    )(x_pad, in_W, in_b, Waug)


def _post_ln(s_ref, xr_ref, h_ref, cb_ref, g_ref, b_ref, residual):
    t = xr_ref[...] + s_ref[0] + s_ref[1] + cb_ref[0]
    m = jnp.mean(t, axis=1, keepdims=True)
    d = t - m
    v = jnp.mean(d * d, axis=1, keepdims=True)
    hn = d * lax.rsqrt(v + 1e-5) * g_ref[0] + b_ref[0]
    hn = jnp.maximum(hn, 0.0)
    if residual:
        hn = hn + h_ref[...]
    return hn


def _post_xr(S, xr_root, h_prev, conv_b, ln_g, ln_b, Waug, l):
    def body(s_ref, xr_ref, h_ref, cb_ref, g_ref, b_ref, wa_ref,
             hout_ref, xrsc_ref, xrroot_ref):
        hn = _post_ln(s_ref, xr_ref, h_ref, cb_ref, g_ref, b_ref,
                      residual=(l > 0))
        hout_ref[...] = hn
        _slabs(hn, wa_ref, xrsc_ref, xrroot_ref)
    return pl.pallas_call(
        body,
        grid=(NBLK,),
        out_shape=(jax.ShapeDtypeStruct((NP, H), jnp.float32),
                   jax.ShapeDtypeStruct((R, NP, H), jnp.float32),
                   jax.ShapeDtypeStruct((NP, H), jnp.float32)),
        in_specs=[
            pl.BlockSpec((NC, BM, H), lambda n: (0, n, 0)),
            pl.BlockSpec((BM, H), lambda n: (n, 0)),
            pl.BlockSpec((BM, H), lambda n: (n, 0)),
            pl.BlockSpec((1, 1, H), lambda n: (l, 0, 0)),
            pl.BlockSpec((1, 1, H), lambda n: (l, 0, 0)),
            pl.BlockSpec((1, 1, H), lambda n: (l, 0, 0)),
            pl.BlockSpec((1, R1, H, H), lambda n: (l + 1, 0, 0, 0)),
        ],
        out_specs=(pl.BlockSpec((BM, H), lambda n: (n, 0)),
                   pl.BlockSpec((R, BM, H), lambda n: (0, n, 0)),
                   pl.BlockSpec((BM, H), lambda n: (n, 0))),
    )(S, xr_root, h_prev, conv_b, ln_g, ln_b, Waug)


def _post_mlp(S, xr_root, h_prev, conv_b, ln_g, ln_b, hW1, hb1, hW2, hb2, l):
    def body(s_ref, xr_ref, h_ref, cb_ref, g_ref, b_ref,
             w1_ref, b1_ref, w2_ref, b2_ref, out_ref):
        hn = _post_ln(s_ref, xr_ref, h_ref, cb_ref, g_ref, b_ref,
                      residual=True)
        h2 = jnp.dot(hn, w1_ref[...],
                     preferred_element_type=jnp.float32) + b1_ref[...]
        h2 = jnp.maximum(h2, 0.0)
        out_ref[...] = jnp.dot(h2, w2_ref[...],
                               preferred_element_type=jnp.float32) + b2_ref[...]
    return pl.pallas_call(
        body,
        grid=(NBLK,),
        out_shape=jax.ShapeDtypeStruct((NP, OUT), jnp.float32),
        in_specs=[
            pl.BlockSpec((NC, BM, H), lambda n: (0, n, 0)),
            pl.BlockSpec((BM, H), lambda n: (n, 0)),
            pl.BlockSpec((BM, H), lambda n: (n, 0)),
            pl.BlockSpec((1, 1, H), lambda n: (l, 0, 0)),
            pl.BlockSpec((1, 1, H), lambda n: (l, 0, 0)),
            pl.BlockSpec((1, 1, H), lambda n: (l, 0, 0)),
            pl.BlockSpec((H, H), lambda n: (0, 0)),
            pl.BlockSpec((1, H), lambda n: (0, 0)),
            pl.BlockSpec((H, OUT), lambda n: (0, 0)),
            pl.BlockSpec((1, OUT), lambda n: (0, 0)),
        ],
        out_specs=pl.BlockSpec((BM, OUT), lambda n: (n, 0)),
    )(S, xr_root, h_prev, conv_b, ln_g, ln_b, hW1, hb1, hW2, hb2)


def _edge_layout(a):
    a = a.reshape(NW, E // NW)
    a = jnp.pad(a, ((0, 0), (0, EPT - E // NW)))
    return a.reshape(EROWS, CHUNK)


def kernel(x, edge_index, edge_type, in_W, in_b, bases, comp, root_w,
           conv_b, ln_g, ln_b, hW1, hb1, hW2, hb2):
    src = edge_index[0]
    dst = edge_index[1]
    et = edge_type
    g2d = _edge_layout(et * NP + src)
    dst2d = _edge_layout(dst)
    key2d = _edge_layout(et * NP + dst)
    valid = jnp.ones((NW, E // NW), jnp.float32)
    valid1d = jnp.pad(valid, ((0, 0), (0, EPT - E // NW))).reshape(EP)

    w1d = _count_weights(key2d, valid1d)

    Waug = _combine_weights(bases, comp, root_w, jnp.asarray(_PMAT))
    x_pad = jnp.pad(x, ((0, NP - N), (0, 0)))
    h, xr_sc, xr_root = _in_xr(x_pad, in_W, in_b.reshape(1, H), Waug)

    cb3 = conv_b.reshape(L, 1, H)
    g3 = ln_g.reshape(L, 1, H)
    b3 = ln_b.reshape(L, 1, H)
    out = None
    for l in range(L):
        S = _aggregate(xr_sc.reshape(R * NP, H), g2d, dst2d, w1d)
        if l < L - 1:
            h, xr_sc, xr_root = _post_xr(S, xr_root, h, cb3, g3, b3, Waug, l)
        else:
            out = _post_mlp(S, xr_root, h, cb3, g3, b3, hW1,
                            hb1.reshape(1, H), hW2, hb2.reshape(1, OUT), l)
    return out[:N]


# speedup vs baseline: 17.2981x; 1.0005x over previous
import functools

import jax
import jax.numpy as jnp
from jax import lax
from jax.experimental import pallas as pl
from jax.experimental.pallas import tpu as pltpu
from jax.experimental.pallas import tpu_sc as plsc

N = 10000
E = 160000
D = 128
H = 128
OUT = 64
R = 10
NB = 4
L = 3

NP = 10240
R1 = R + 1
CNTP = R * NP
NC = 2
NS = 16
NW = NC * NS
EPT = 5120
CHUNK = 128
NCH = EPT // CHUNK
EP = NW * EPT
EROWS = EP // CHUNK
CROWS = EP // NS // CHUNK
NROWS_W = NP // NS // CHUNK
CPT = CNTP // NS

BM = 256
NBLK = NP // BM

_MESH = plsc.VectorSubcoreMesh(
    core_axis_name="c", subcore_axis_name="s", num_cores=NC, num_subcores=NS)


def _zero_rows(ref, nrows):
    def body(i, _):
        for q in range(CHUNK // 16):
            ref[i, pl.ds(q * 16, 16)] = jnp.zeros((16,), jnp.float32)
        return 0
    lax.fori_loop(0, nrows, body, 0, unroll=False)


@functools.partial(
    pl.kernel,
    out_type=jax.ShapeDtypeStruct((EP,), jnp.float32),
    mesh=_MESH,
    scratch_types=[
        pltpu.VMEM((CROWS, CHUNK), jnp.int32),
        pltpu.VMEM((EP // NS,), jnp.float32),
        pltpu.VMEM((NCH, CHUNK), jnp.int32),
        pltpu.VMEM((EPT,), jnp.float32),
        pltpu.VMEM((EPT,), jnp.float32),
        pltpu.VMEM((EPT,), jnp.float32),
        pltpu.VMEM((CPT,), jnp.float32),
        pltpu.VMEM_SHARED((CNTP,), jnp.float32),
    ],
)
def _count_weights(key_hbm, valid_hbm, w_hbm,
                   key_a, valid_a, key_b, valid_b, cnt_g, w_v, zbuf, cnt_sh):
    c = lax.axis_index("c")
    s = lax.axis_index("s")
    wid = c * NS + s
    def zbody(i, _):
        zbuf[pl.ds(i * 16, 16)] = jnp.zeros((16,), jnp.float32)
        return 0
    lax.fori_loop(0, CPT // 16, zbody, 0, unroll=False)
    pltpu.sync_copy(zbuf, cnt_sh.at[pl.ds(s * CPT, CPT)])
    plsc.subcore_barrier()
    pltpu.sync_copy(key_hbm.at[pl.ds(s * CROWS, CROWS)], key_a)
    pltpu.sync_copy(valid_hbm.at[pl.ds(s * (EP // NS), EP // NS)], valid_a)

    def cbody(j, _):
        pltpu.sync_copy(valid_a.at[pl.ds(j * CHUNK, CHUNK)],
                        cnt_sh.at[key_a.at[j]], add=True)
        return 0
    lax.fori_loop(0, CROWS, cbody, 0, unroll=False)
    plsc.subcore_barrier()
    pltpu.sync_copy(key_hbm.at[pl.ds(wid * NCH, NCH)], key_b)
    pltpu.sync_copy(valid_hbm.at[pl.ds(wid * EPT, EPT)], valid_b)

    def gbody(j, _):
        pltpu.sync_copy(cnt_sh.at[key_b.at[j]],
                        cnt_g.at[pl.ds(j * CHUNK, CHUNK)])
        return 0
    lax.fori_loop(0, NCH, gbody, 0, unroll=False)

    def wbody(i, _):
        sl = pl.ds(i * 16, 16)
        w_v[sl] = valid_b[sl] / jnp.maximum(cnt_g[sl], 1.0)
        return 0
    lax.fori_loop(0, EPT // 16, wbody, 0, unroll=False)
    pltpu.sync_copy(w_v, w_hbm.at[pl.ds(wid * EPT, EPT)])


@functools.partial(
    pl.kernel,
    out_type=jax.ShapeDtypeStruct((NC, NP, H), jnp.float32),
    mesh=_MESH,
    scratch_types=[
        pltpu.VMEM((NCH, CHUNK), jnp.int32),
        pltpu.VMEM((NCH, CHUNK), jnp.int32),
        pltpu.VMEM((EPT,), jnp.float32),
        pltpu.VMEM((CHUNK, H), jnp.float32),
        pltpu.VMEM((CHUNK, H), jnp.float32),
        pltpu.VMEM_SHARED((NP, H), jnp.float32),
        pltpu.SemaphoreType.DMA,
        pltpu.SemaphoreType.DMA,
    ],
    compiler_params=pltpu.CompilerParams(needs_layout_passes=False),
)
def _aggregate(xr_hbm, g_hbm, dst_hbm, w_hbm, out_hbm,
               g_v, dst_v, w_v, in0, in1, acc_sh, sem, sem_s):
    c = lax.axis_index("c")
    s = lax.axis_index("s")
    wid = c * NS + s
    _zero_rows(in0, CHUNK)
    for t in range(NROWS_W):
        pltpu.sync_copy(in0, acc_sh.at[pl.ds((s * NROWS_W + t) * CHUNK, CHUNK)])
    plsc.subcore_barrier()
    pltpu.sync_copy(g_hbm.at[pl.ds(wid * NCH, NCH)], g_v)
    pltpu.sync_copy(dst_hbm.at[pl.ds(wid * NCH, NCH)], dst_v)
    pltpu.sync_copy(w_hbm.at[pl.ds(wid * EPT, EPT)], w_v)

    bufs = (in0, in1)
    NSPLIT = 4
    HC = CHUNK // NSPLIT

    def start_gather(j, buf):
        for p in range(NSPLIT):
            pltpu.async_copy(xr_hbm.at[g_v.at[j, pl.ds(p * HC, HC)]],
                             buf.at[pl.ds(p * HC, HC)], sem)

    def wait_gather(j, buf):
        for p in range(NSPLIT):
            pltpu.make_async_copy(xr_hbm.at[g_v.at[j, pl.ds(p * HC, HC)]],
                                  buf.at[pl.ds(p * HC, HC)], sem).wait()

    start_gather(0, in0)

    def scale(j, buf):
        jbase = j * CHUNK

        def grp_body(t, _):
            wg = w_v[pl.ds(jbase + t * 16, 16)]
            for r in range(16):
                ws = wg[r]
                row = t * 16 + r
                for q in range(H // 16):
                    sl = pl.ds(q * 16, 16)
                    buf[row, sl] = buf[row, sl] * ws
            return 0
        lax.fori_loop(0, CHUNK // 16, grp_body, 0, unroll=False)

    def pair_body(j2, _):
        for b in range(2):
            j = j2 * 2 + b
            cur, nxt = bufs[b], bufs[1 - b]

            @pl.when(j >= 1)
            def _():
                pltpu.make_async_copy(
                    nxt, acc_sh.at[dst_v.at[j - 1]], sem_s).wait()

            @pl.when(j < NCH - 1)
            def _():
                start_gather(j + 1, nxt)
            wait_gather(j, cur)
            scale(j, cur)
            pltpu.async_copy(cur, acc_sh.at[dst_v.at[j]], sem_s, add=True)
        return 0
    lax.fori_loop(0, NCH // 2, pair_body, 0, unroll=False)
    pltpu.make_async_copy(
        bufs[1], acc_sh.at[dst_v.at[NCH - 1]], sem_s).wait()
    plsc.subcore_barrier()
    for t in range(NROWS_W):
        rs = pl.ds((s * NROWS_W + t) * CHUNK, CHUNK)
        pltpu.sync_copy(acc_sh.at[rs], in0)
        pltpu.sync_copy(in0, out_hbm.at[c, rs])


def _combine_weights(bases, comp, root_w):
    def body(comp_ref, bases_ref, root_ref, out_ref):
        for l in range(L):
            for r in range(R):
                acc = comp_ref[l, r, 0] * bases_ref[l, 0]
                for b in range(1, NB):
                    acc = acc + comp_ref[l, r, b] * bases_ref[l, b]
                out_ref[l, r] = acc
            out_ref[l, R] = root_ref[l]
    return pl.pallas_call(
        body,
        out_shape=jax.ShapeDtypeStruct((L, R1, H, H), jnp.float32),
        in_specs=[
            pl.BlockSpec(memory_space=pltpu.SMEM),
            pl.BlockSpec(memory_space=pltpu.VMEM),
            pl.BlockSpec(memory_space=pltpu.VMEM),
        ],
        out_specs=pl.BlockSpec(memory_space=pltpu.VMEM),
    )(comp, bases, root_w)


def _slabs(h0, wa_ref, xrsc_ref, xrroot_ref):
    for r in range(R):
        xrsc_ref[r] = jnp.dot(h0, wa_ref[0, r],
                              preferred_element_type=jnp.float32)
    xrroot_ref[...] = jnp.dot(h0, wa_ref[0, R],
                              preferred_element_type=jnp.float32)


def _in_xr(x_pad, in_W, in_b, Waug):
    def body(x_ref, w_ref, b_ref, wa_ref, h_ref, xrsc_ref, xrroot_ref):
        h0 = jnp.dot(x_ref[...], w_ref[...],
                     preferred_element_type=jnp.float32) + b_ref[...]
        h_ref[...] = h0
        _slabs(h0, wa_ref, xrsc_ref, xrroot_ref)
    return pl.pallas_call(
        body,
        grid=(NBLK,),
        out_shape=(jax.ShapeDtypeStruct((NP, H), jnp.float32),
                   jax.ShapeDtypeStruct((R, NP, H), jnp.float32),
                   jax.ShapeDtypeStruct((NP, H), jnp.float32)),
        in_specs=[
            pl.BlockSpec((BM, D), lambda n: (n, 0)),
            pl.BlockSpec((D, H), lambda n: (0, 0)),
            pl.BlockSpec((1, H), lambda n: (0, 0)),
            pl.BlockSpec((1, R1, H, H), lambda n: (0, 0, 0, 0)),
        ],
        out_specs=(pl.BlockSpec((BM, H), lambda n: (n, 0)),
                   pl.BlockSpec((R, BM, H), lambda n: (0, n, 0)),
                   pl.BlockSpec((BM, H), lambda n: (n, 0))),
    )(x_pad, in_W, in_b, Waug)


def _post_ln(s_ref, xr_ref, h_ref, cb_ref, g_ref, b_ref, residual):
    t = xr_ref[...] + s_ref[0] + s_ref[1] + cb_ref[0]
    m = jnp.mean(t, axis=1, keepdims=True)
    d = t - m
    v = jnp.mean(d * d, axis=1, keepdims=True)
    hn = d * lax.rsqrt(v + 1e-5) * g_ref[0] + b_ref[0]
    hn = jnp.maximum(hn, 0.0)
    if residual:
        hn = hn + h_ref[...]
    return hn


def _post_xr(S, xr_root, h_prev, conv_b, ln_g, ln_b, Waug, l):
    def body(s_ref, xr_ref, h_ref, cb_ref, g_ref, b_ref, wa_ref,
             hout_ref, xrsc_ref, xrroot_ref):
        hn = _post_ln(s_ref, xr_ref, h_ref, cb_ref, g_ref, b_ref,
                      residual=(l > 0))
        hout_ref[...] = hn
        _slabs(hn, wa_ref, xrsc_ref, xrroot_ref)
    return pl.pallas_call(
        body,
        grid=(NBLK,),
        out_shape=(jax.ShapeDtypeStruct((NP, H), jnp.float32),
                   jax.ShapeDtypeStruct((R, NP, H), jnp.float32),
                   jax.ShapeDtypeStruct((NP, H), jnp.float32)),
        in_specs=[
            pl.BlockSpec((NC, BM, H), lambda n: (0, n, 0)),
            pl.BlockSpec((BM, H), lambda n: (n, 0)),
            pl.BlockSpec((BM, H), lambda n: (n, 0)),
            pl.BlockSpec((1, 1, H), lambda n: (l, 0, 0)),
            pl.BlockSpec((1, 1, H), lambda n: (l, 0, 0)),
            pl.BlockSpec((1, 1, H), lambda n: (l, 0, 0)),
            pl.BlockSpec((1, R1, H, H), lambda n: (l + 1, 0, 0, 0)),
        ],
        out_specs=(pl.BlockSpec((BM, H), lambda n: (n, 0)),
                   pl.BlockSpec((R, BM, H), lambda n: (0, n, 0)),
                   pl.BlockSpec((BM, H), lambda n: (n, 0))),
    )(S, xr_root, h_prev, conv_b, ln_g, ln_b, Waug)


def _post_mlp(S, xr_root, h_prev, conv_b, ln_g, ln_b, hW1, hb1, hW2, hb2, l):
    def body(s_ref, xr_ref, h_ref, cb_ref, g_ref, b_ref,
             w1_ref, b1_ref, w2_ref, b2_ref, out_ref):
        hn = _post_ln(s_ref, xr_ref, h_ref, cb_ref, g_ref, b_ref,
                      residual=True)
        h2 = jnp.dot(hn, w1_ref[...],
                     preferred_element_type=jnp.float32) + b1_ref[...]
        h2 = jnp.maximum(h2, 0.0)
        out_ref[...] = jnp.dot(h2, w2_ref[...],
                               preferred_element_type=jnp.float32) + b2_ref[...]
    return pl.pallas_call(
        body,
        grid=(NBLK,),
        out_shape=jax.ShapeDtypeStruct((NP, OUT), jnp.float32),
        in_specs=[
            pl.BlockSpec((NC, BM, H), lambda n: (0, n, 0)),
            pl.BlockSpec((BM, H), lambda n: (n, 0)),
            pl.BlockSpec((BM, H), lambda n: (n, 0)),
            pl.BlockSpec((1, 1, H), lambda n: (l, 0, 0)),
            pl.BlockSpec((1, 1, H), lambda n: (l, 0, 0)),
            pl.BlockSpec((1, 1, H), lambda n: (l, 0, 0)),
            pl.BlockSpec((H, H), lambda n: (0, 0)),
            pl.BlockSpec((1, H), lambda n: (0, 0)),
            pl.BlockSpec((H, OUT), lambda n: (0, 0)),
            pl.BlockSpec((1, OUT), lambda n: (0, 0)),
        ],
        out_specs=pl.BlockSpec((BM, OUT), lambda n: (n, 0)),
    )(S, xr_root, h_prev, conv_b, ln_g, ln_b, hW1, hb1, hW2, hb2)


def _edge_layout(a):
    a = a.reshape(NW, E // NW)
    a = jnp.pad(a, ((0, 0), (0, EPT - E // NW)))
    return a.reshape(EROWS, CHUNK)


def kernel(x, edge_index, edge_type, in_W, in_b, bases, comp, root_w,
           conv_b, ln_g, ln_b, hW1, hb1, hW2, hb2):
    src = edge_index[0]
    dst = edge_index[1]
    et = edge_type
    g2d = _edge_layout(et * NP + src)
    dst2d = _edge_layout(dst)
    key2d = _edge_layout(et * NP + dst)
    valid = jnp.ones((NW, E // NW), jnp.float32)
    valid1d = jnp.pad(valid, ((0, 0), (0, EPT - E // NW))).reshape(EP)

    w1d = _count_weights(key2d, valid1d)

    Waug = _combine_weights(bases, comp, root_w)
    x_pad = jnp.pad(x, ((0, NP - N), (0, 0)))
    h, xr_sc, xr_root = _in_xr(x_pad, in_W, in_b.reshape(1, H), Waug)

    cb3 = conv_b.reshape(L, 1, H)
    g3 = ln_g.reshape(L, 1, H)
    b3 = ln_b.reshape(L, 1, H)
    out = None
    for l in range(L):
        S = _aggregate(xr_sc.reshape(R * NP, H), g2d, dst2d, w1d)
        if l < L - 1:
            h, xr_sc, xr_root = _post_xr(S, xr_root, h, cb3, g3, b3, Waug, l)
        else:
            out = _post_mlp(S, xr_root, h, cb3, g3, b3, hW1,
                            hb1.reshape(1, H), hW2, hb2.reshape(1, OUT), l)
    return out[:N]
